# baseline (speedup 1.0000x reference)
import functools

import numpy as np

import jax
import jax.numpy as jnp
from jax import lax
from jax.experimental import pallas as pl
from jax.experimental.pallas import tpu as pltpu
from jax.experimental.pallas import tpu_sc as plsc

N = 10000
E = 320000
D = 128

NC = 2
NS = 16
NW = NC * NS
L = 16

NPW = 320
NPAD = NW * NPW
EPW = E // NW
KB = 80
NB_CH = EPW // KB
CS = 2000
ND_CH = EPW // CS
G = 64
WLCAP = CS + G


_PERM = np.zeros(D, np.int64)
for _k in range(D // 32):
    for _i in range(16):
        _PERM[32 * _k + 2 * _i] = 32 * _k + _i
        _PERM[32 * _k + 2 * _i + 1] = 32 * _k + 16 + _i


def _gelu(x):
    return x * 0.5 * (1.0 + lax.erf(x * 0.7071067811865476))


def _eff_w(p):
    V, g, b = p["v"], p["g"], p["b"]
    norm = jnp.sqrt(jnp.sum(V * V, axis=1, keepdims=True))
    return (g * V / norm), b


def _stage_a_body(feat_ref, w1_ref, w2_ref, wh_ref, bh_ref, ws_ref, bs_ref,
                  tb1_ref, tb2_ref, th_ref, s_ref):
    f = feat_ref[...]
    tb1_ref[...] = jnp.dot(f, w1_ref[...],
                           preferred_element_type=jnp.float32).astype(jnp.bfloat16)
    tb2_ref[...] = jnp.dot(f, w2_ref[...],
                           preferred_element_type=jnp.float32).astype(jnp.bfloat16)
    th_ref[...] = _gelu(jnp.dot(f, wh_ref[...], preferred_element_type=jnp.float32)
                        + bh_ref[...])
    s_ref[...] = jnp.dot(f, ws_ref[...], preferred_element_type=jnp.float32) + bs_ref[...]


def _stage_a(feat, w1cat, w2cat, whcat, bhcat, wself, bself):
    blk = 2000
    grid = (N // blk,)
    full = lambda shape: pl.BlockSpec(shape, lambda i: (0, 0))
    return pl.pallas_call(
        _stage_a_body,
        grid=grid,
        in_specs=[
            pl.BlockSpec((blk, D), lambda i: (i, 0)),
            full((D, 2 * D)), full((D, 2 * D)), full((D, 2 * D)),
            full((1, 2 * D)), full((D, D)), full((1, D)),
        ],
        out_specs=[
            pl.BlockSpec((blk, 2 * D), lambda i: (i, 0)),
            pl.BlockSpec((blk, 2 * D), lambda i: (i, 0)),
            pl.BlockSpec((blk, 2 * D), lambda i: (i, 0)),
            pl.BlockSpec((blk, D), lambda i: (i, 0)),
        ],
        out_shape=[
            jax.ShapeDtypeStruct((N, 2 * D), jnp.bfloat16),
            jax.ShapeDtypeStruct((N, 2 * D), jnp.bfloat16),
            jax.ShapeDtypeStruct((N, 2 * D), jnp.float32),
            jax.ShapeDtypeStruct((N, D), jnp.float32),
        ],
    )(feat, w1cat, w2cat, whcat, bhcat, wself, bself)


def _stage_b_kernel(src_hbm, dst_hbm, tb1_hbm, tb2_hbm, out_hbm,
                    sidx, didx, abuf, bbuf, obuf, sema0, semb0, sema1, semb1):
    wid = lax.axis_index("s") * NC + lax.axis_index("c")
    base = wid * EPW
    gsems = ((sema0, semb0), (sema1, semb1))

    def issue(j, t):
        off = base + j * KB
        pltpu.sync_copy(src_hbm.at[pl.ds(off, KB)], sidx.at[t])
        pltpu.sync_copy(dst_hbm.at[pl.ds(off, KB)], didx.at[t])
        pltpu.async_copy(tb1_hbm.at[sidx.at[t]], abuf.at[t], gsems[t][0])
        pltpu.async_copy(tb2_hbm.at[didx.at[t]], bbuf.at[t], gsems[t][1])

    def wait_slot(t):
        pltpu.make_async_copy(tb1_hbm.at[sidx.at[t]], abuf.at[t],
                              gsems[t][0]).wait()
        pltpu.make_async_copy(tb2_hbm.at[didx.at[t]], bbuf.at[t],
                              gsems[t][1]).wait()

    def compute_out(j, t):
        def row(k, c2):
            for r in range(8):
                a = plsc.bitcast(abuf[t, k, pl.ds(r * L, L)], jnp.bfloat16)
                b = plsc.bitcast(bbuf[t, k, pl.ds(r * L, L)], jnp.bfloat16)
                v = a * b if r < 4 else a + b
                obuf[t, k, pl.ds(r * L, L)] = plsc.bitcast(v, jnp.int32)
            return c2
        lax.fori_loop(0, KB, row, 0, unroll=2)
        pltpu.sync_copy(obuf.at[t], out_hbm.at[pl.ds(base + j * KB, KB)])

    issue(0, 0)

    @pl.loop(0, NB_CH - 1, step=2)
    def chunk_pair(j):
        wait_slot(0)
        issue(j + 1, 1)
        compute_out(j, 0)
        wait_slot(1)

        @pl.when(j + 2 < NB_CH)
        def _():
            issue(j + 2, 0)
        compute_out(j + 1, 1)

    wait_slot(0)
    compute_out(NB_CH - 1, 0)


def _stage_b(src, dst, tb1, tb2):
    mesh = plsc.VectorSubcoreMesh(core_axis_name="c", subcore_axis_name="s")
    f = pl.kernel(
        _stage_b_kernel,
        out_type=jax.ShapeDtypeStruct((E, D), jnp.int32),
        mesh=mesh,
        compiler_params=pltpu.CompilerParams(needs_layout_passes=False),
        scratch_types=[
            pltpu.VMEM((2, KB), jnp.int32),
            pltpu.VMEM((2, KB), jnp.int32),
            pltpu.VMEM((2, KB, D), jnp.int32),
            pltpu.VMEM((2, KB, D), jnp.int32),
            pltpu.VMEM((2, KB, D), jnp.int32),
            pltpu.SemaphoreType.DMA,
            pltpu.SemaphoreType.DMA,
            pltpu.SemaphoreType.DMA,
            pltpu.SemaphoreType.DMA,
        ],
    )
    return f(src, dst, tb1, tb2)


def _stage_c_body(ab_ref, wm_ref, btot_ref, wo_ref, ab2_ref, out_ref):
    ab = ab_ref[...].astype(jnp.float32)
    prod = ab[:, :D]
    gsum = ab[:, D:]
    pre = jnp.dot(prod, wm_ref[...], preferred_element_type=jnp.float32) + gsum + btot_ref[...]
    x = _gelu(pre)
    y = jnp.dot(x, wo_ref[...], preferred_element_type=jnp.float32) + ab2_ref[...]
    out_ref[...] = jnp.where(y > 0, y, 0.2 * y)


def _stage_c(ab, wmul, btot, wout2, bout2):
    blk = 3200
    grid = (E // blk,)
    return pl.pallas_call(
        _stage_c_body,
        grid=grid,
        in_specs=[
            pl.BlockSpec((blk, 2 * D), lambda i: (i, 0)),
            pl.BlockSpec((D, D), lambda i: (0, 0)),
            pl.BlockSpec((1, D), lambda i: (0, 0)),
            pl.BlockSpec((D, 1), lambda i: (0, 0)),
            pl.BlockSpec((1, 1), lambda i: (0, 0)),
        ],
        out_specs=pl.BlockSpec((blk, 1), lambda i: (i, 0)),
        out_shape=jax.ShapeDtypeStruct((E, 1), jnp.float32),
    )(ab, wmul, btot, wout2, bout2)


NCH = E // CS


def _stage_d_kernel(src_hbm, dst_hbm, e_hbm, th_hbm,
                    maxout, sumout, degout,
                    maxacc, sumacc, degacc,
                    sbuf0, dbuf0, ebuf0, sbuf1, dbuf1, ebuf1,
                    wl_src, wl_row, wl_e, gbuf, csem, gsem):
    sid = lax.axis_index("s")
    wid = lax.axis_index("c") * NS + sid
    lo = wid * NPW
    iota = lax.iota(jnp.int32, L)

    zf = jnp.zeros((L,), jnp.float32)
    zi = jnp.zeros((L,), jnp.int32)
    ninf = jnp.full((L,), -1e30, jnp.float32)

    def init_rows(r, c):
        for cc in range(8):
            maxacc[r, pl.ds(cc * L, L)] = ninf
            sumacc[r, pl.ds(cc * L, L)] = zf
        return c
    lax.fori_loop(0, NPW, init_rows, 0, unroll=False)

    def init_deg(g, c):
        degacc[pl.ds(g * L, L)] = zf
        return c
    lax.fori_loop(0, NPW // L, init_deg, 0, unroll=False)

    def init_wl(g, c):
        wl_src[pl.ds(g * L, L)] = zi
        return c
    lax.fori_loop(0, WLCAP // L, init_wl, 0, unroll=False)

    bufs0 = (sbuf0, dbuf0, ebuf0)
    bufs1 = (sbuf1, dbuf1, ebuf1)

    def issue_chunk(ch, bufs):
        off = ch * CS
        pltpu.async_copy(src_hbm.at[pl.ds(off, CS)], bufs[0], csem)
        pltpu.async_copy(dst_hbm.at[pl.ds(off, CS)], bufs[1], csem)
        pltpu.async_copy(e_hbm.at[pl.ds(off, CS)], bufs[2], csem)

    def wait_chunk(bufs):
        pltpu.make_async_copy(src_hbm.at[pl.ds(0, CS)], bufs[0], csem).wait()
        pltpu.make_async_copy(src_hbm.at[pl.ds(0, CS)], bufs[1], csem).wait()
        pltpu.make_async_copy(e_hbm.at[pl.ds(0, CS)], bufs[2], csem).wait()

    colvs = [jnp.full((L,), cc * L, jnp.int32) + iota for cc in range(8)]

    def edge_work(wl_base, j):
        jv = jnp.full((L,), wl_base + j, jnp.int32)
        rowv = plsc.load_gather(wl_row, [jv])
        evv = plsc.load_gather(wl_e, [jv])
        olds = [plsc.load_gather(maxacc, [rowv, colvs[cc]]) for cc in range(8)]
        gvs = [gbuf[j, pl.ds(cc * L, L)] for cc in range(8)]
        g2s = [gbuf[j, pl.ds(D + cc * L, L)] for cc in range(8)]
        for cc in range(8):
            plsc.store_scatter(maxacc, [rowv, colvs[cc]],
                               jnp.maximum(olds[cc], evv * gvs[cc]))
        for cc in range(8):
            plsc.addupdate_scatter(sumacc, [rowv, colvs[cc]], evv * g2s[cc])

    def do_chunk(bufs, carry):
        sb, db, eb = bufs

        def scang(g, wpv):
            dv = db[pl.ds(g * L, L)]
            m = (dv >= lo) & (dv < lo + NPW)
            sv = sb[pl.ds(g * L, L)]
            ev = eb[pl.ds(g * L, L)]
            rv = dv - lo
            cum = plsc.cumsum(jnp.where(m, 1, 0))
            cntv = plsc.all_reduce_population_count(m)
            posv = wpv + cum - 1
            plsc.store_scatter(wl_src, [posv], sv, mask=m)
            plsc.store_scatter(wl_row, [posv], rv, mask=m)
            plsc.store_scatter(wl_e, [posv], ev, mask=m)
            plsc.addupdate_scatter(degacc, [rv],
                                   jnp.ones((L,), jnp.float32), mask=m)
            return wpv + cntv

        totalv = lax.fori_loop(0, CS // L, scang, carry, unroll=2)
        total = totalv[0]
        nsub = total // G

        def sub(q, c):
            pltpu.async_copy(th_hbm.at[wl_src.at[pl.ds(q * G, G)]],
                             gbuf, gsem).wait()

            def edge(j, c2):
                edge_work(q * G, j)
                return c2
            lax.fori_loop(0, G, edge, 0, unroll=2)
            return c
        lax.fori_loop(0, nsub, sub, 0, unroll=False)

        rem = total - nsub * G
        s0 = nsub * G
        for t in range(G // L):
            o = t * L
            a = wl_src[pl.ds(s0 + o, L)]
            b = wl_row[pl.ds(s0 + o, L)]
            cvals = wl_e[pl.ds(s0 + o, L)]
            wl_src[pl.ds(o, L)] = a
            wl_row[pl.ds(o, L)] = b
            wl_e[pl.ds(o, L)] = cvals
        return totalv - nsub * G

    issue_chunk(0, bufs0)

    def chunk_pair(ch, carry_in):
        wait_chunk(bufs0)
        issue_chunk(ch + 1, bufs1)
        c1 = do_chunk(bufs0, carry_in)
        wait_chunk(bufs1)

        @pl.when(ch + 2 < NCH)
        def _():
            issue_chunk(ch + 2, bufs0)
        return do_chunk(bufs1, c1)

    carryv = pl.loop(0, NCH, step=2,
                     init_carry=jnp.zeros((L,), jnp.int32))(chunk_pair)
    carry = carryv[0]

    pltpu.async_copy(th_hbm.at[wl_src.at[pl.ds(0, G)]], gbuf, gsem).wait()

    def edge_fin(j, c):
        edge_work(0, j)
        return c
    lax.fori_loop(0, carry, edge_fin, 0, unroll=False)

    pltpu.sync_copy(maxacc, maxout.at[pl.ds(lo, NPW)])
    pltpu.sync_copy(sumacc, sumout.at[pl.ds(lo, NPW)])
    pltpu.sync_copy(degacc, degout.at[pl.ds(lo, NPW)])


def _stage_d(src, dst, e, th):
    mesh = plsc.VectorSubcoreMesh(core_axis_name="c", subcore_axis_name="s")
    f = pl.kernel(
        _stage_d_kernel,
        out_type=[
            jax.ShapeDtypeStruct((NPAD, D), jnp.float32),
            jax.ShapeDtypeStruct((NPAD, D), jnp.float32),
            jax.ShapeDtypeStruct((NPAD,), jnp.float32),
        ],
        mesh=mesh,
        compiler_params=pltpu.CompilerParams(needs_layout_passes=False),
        scratch_types=[
            pltpu.VMEM((NPW, D), jnp.float32),
            pltpu.VMEM((NPW, D), jnp.float32),
            pltpu.VMEM((NPW,), jnp.float32),
            pltpu.VMEM((CS,), jnp.int32),
            pltpu.VMEM((CS,), jnp.int32),
            pltpu.VMEM((CS,), jnp.float32),
            pltpu.VMEM((CS,), jnp.int32),
            pltpu.VMEM((CS,), jnp.int32),
            pltpu.VMEM((CS,), jnp.float32),
            pltpu.VMEM((WLCAP,), jnp.int32),
            pltpu.VMEM((WLCAP,), jnp.int32),
            pltpu.VMEM((WLCAP,), jnp.float32),
            pltpu.VMEM((G, 2 * D), jnp.float32),
            pltpu.SemaphoreType.DMA,
            pltpu.SemaphoreType.DMA,
        ],
    )
    return f(src, dst, e, th)


def _stage_e_body(mx_ref, sm_ref, deg_ref, s_ref, wn_ref, bn_ref,
                  wn2_ref, bn2_ref, wm1_ref, bm1_ref, wm2_ref, bm2_ref,
                  out_ref):
    dg = deg_ref[...]
    neigh = jnp.where(dg > 0, mx_ref[...], 0.0)
    neigh2 = sm_ref[...] / jnp.maximum(dg, 1.0)
    rst = (s_ref[...]
           + jnp.dot(neigh, wn_ref[...], preferred_element_type=jnp.float32) + bn_ref[...]
           + jnp.dot(neigh2, wn2_ref[...], preferred_element_type=jnp.float32) + bn2_ref[...])
    rst = rst + jnp.dot(_gelu(rst), wm1_ref[...], preferred_element_type=jnp.float32) + bm1_ref[...]
    rst = rst + jnp.dot(_gelu(rst), wm2_ref[...], preferred_element_type=jnp.float32) + bm2_ref[...]
    out_ref[...] = rst


def _stage_e(mx, sm, deg, s, wn, bn, wn2, bn2, wm1, bm1, wm2, bm2):
    blk = 2000
    grid = (N // blk,)
    wfull = lambda: pl.BlockSpec((D, D), lambda i: (0, 0))
    bfull = lambda: pl.BlockSpec((1, D), lambda i: (0, 0))
    return pl.pallas_call(
        _stage_e_body,
        grid=grid,
        in_specs=[
            pl.BlockSpec((blk, D), lambda i: (i, 0)),
            pl.BlockSpec((blk, D), lambda i: (i, 0)),
            pl.BlockSpec((blk, 1), lambda i: (i, 0)),
            pl.BlockSpec((blk, D), lambda i: (i, 0)),
            wfull(), bfull(), wfull(), bfull(),
            wfull(), bfull(), wfull(), bfull(),
        ],
        out_specs=pl.BlockSpec((blk, D), lambda i: (i, 0)),
        out_shape=jax.ShapeDtypeStruct((N, D), jnp.float32),
    )(mx, sm, deg, s, wn, bn, wn2, bn2, wm1, bm1, wm2, bm2)


_DBG_B = False
_DBG_D = False

def kernel(feat, edge_index, params):
    wsrc, bsrc = _eff_w(params["atten_src"])
    wdst, bdst = _eff_w(params["atten_dst"])
    wsub, bsub = _eff_w(params["atten_sub"])
    wmul, bmul = _eff_w(params["atten_mul"])
    wout, bout = _eff_w(params["atten_out"])
    wp, bp = _eff_w(params["fc_pool"])
    wp2, bp2 = _eff_w(params["fc_pool2"])
    wself, bself = _eff_w(params["fc_self"])
    wn, bn = _eff_w(params["fc_neigh"])
    wn2, bn2 = _eff_w(params["fc_neigh2"])
    btot = (bsrc + bdst + bsub + bmul)[None, :]

    eye = jnp.eye(D, dtype=jnp.float32)
    w1cat = jnp.concatenate([eye, (wsrc + wsub).T], axis=1)
    w2cat = jnp.concatenate([eye, (wdst - wsub).T], axis=1)
    whcat = jnp.concatenate([wp.T, wp2.T], axis=1)
    bhcat = jnp.concatenate([bp, bp2])[None, :]

    tb1, tb2, th, s = _stage_a(feat, w1cat, w2cat, whcat, bhcat,
                               wself.T, bself[None, :])

    src = edge_index[0]
    dst = edge_index[1]
    tb1 = lax.bitcast_convert_type(tb1.reshape(N, D, 2), jnp.int32)
    tb2 = lax.bitcast_convert_type(tb2.reshape(N, D, 2), jnp.int32)
    if _DBG_B:
        ab = jnp.concatenate([tb1[src, :D] * tb2[dst, :D],
                              tb1[src, D:] + tb2[dst, D:]], axis=1)
    else:
        ab = _stage_b(src, dst, tb1, tb2)

    ab16 = lax.bitcast_convert_type(ab, jnp.bfloat16).reshape(E, 2 * D)
    e2 = _stage_c(ab16, wmul.T, btot, wout.T, bout[None, None, 0])
    e = e2.reshape(E)

    if _DBG_D:
        mx = jax.ops.segment_max(e[:, None] * th[src, :D], dst, num_segments=NPAD)
        mx = jnp.where(jnp.isfinite(mx), mx, -1e30)
        sm = jax.ops.segment_sum(e[:, None] * th[src, D:], dst, num_segments=NPAD)
        deg = jax.ops.segment_sum(jnp.ones((E,), jnp.float32), dst, num_segments=NPAD)
    else:
        mx, sm, deg = _stage_d(src, dst, e, th)
    mx = mx[:N]
    sm = sm[:N]
    deg = deg[:N].reshape(N, 1)

    wmlp1, bmlp1 = _eff_w(params["out_mlp"][0])
    wmlp2, bmlp2 = _eff_w(params["out_mlp"][1])
    rst = _stage_e(mx, sm, deg, s, wn.T, bn[None, :], wn2.T, bn2[None, :],
                   wmlp1.T, bmlp1[None, :], wmlp2.T, bmlp2[None, :])
    return rst


# speedup vs baseline: 1.9595x; 1.9595x over previous
import functools

import numpy as np

import jax
import jax.numpy as jnp
from jax import lax
from jax.experimental import pallas as pl
from jax.experimental.pallas import tpu as pltpu
from jax.experimental.pallas import tpu_sc as plsc

N = 10000
E = 320000
D = 128

NC = 2
NS = 16
NW = NC * NS
L = 16

NPW = 320
NPAD = NW * NPW
EPW = E // NW
KB = 80
NB_CH = EPW // KB
CS = 2000
ND_CH = EPW // CS
G = 64
WLCAP = CS + G


def _gelu(x):
    return x * 0.5 * (1.0 + lax.erf(x * 0.7071067811865476))


def _eff_w(p):
    V, g, b = p["v"], p["g"], p["b"]
    norm = jnp.sqrt(jnp.sum(V * V, axis=1, keepdims=True))
    return (g * V / norm), b


def _stage_a_body(feat_ref, w1_ref, w2_ref, wh_ref, bh_ref, ws_ref, bs_ref,
                  tb1_ref, tb2_ref, th_ref, s_ref):
    f = feat_ref[...]
    tb1_ref[...] = jnp.dot(f, w1_ref[...], preferred_element_type=jnp.float32)
    tb2_ref[...] = jnp.dot(f, w2_ref[...], preferred_element_type=jnp.float32)
    th_ref[...] = _gelu(jnp.dot(f, wh_ref[...], preferred_element_type=jnp.float32)
                        + bh_ref[...])
    s_ref[...] = jnp.dot(f, ws_ref[...], preferred_element_type=jnp.float32) + bs_ref[...]


def _stage_a(feat, w1cat, w2cat, whcat, bhcat, wself, bself):
    blk = 2000
    grid = (N // blk,)
    full = lambda shape: pl.BlockSpec(shape, lambda i: (0, 0))
    return pl.pallas_call(
        _stage_a_body,
        grid=grid,
        in_specs=[
            pl.BlockSpec((blk, D), lambda i: (i, 0)),
            full((D, 2 * D)), full((D, 2 * D)), full((D, 2 * D)),
            full((1, 2 * D)), full((D, D)), full((1, D)),
        ],
        out_specs=[
            pl.BlockSpec((blk, 2 * D), lambda i: (i, 0)),
            pl.BlockSpec((blk, 2 * D), lambda i: (i, 0)),
            pl.BlockSpec((blk, 2 * D), lambda i: (i, 0)),
            pl.BlockSpec((blk, D), lambda i: (i, 0)),
        ],
        out_shape=[
            jax.ShapeDtypeStruct((N, 2 * D), jnp.float32),
            jax.ShapeDtypeStruct((N, 2 * D), jnp.float32),
            jax.ShapeDtypeStruct((N, 2 * D), jnp.float32),
            jax.ShapeDtypeStruct((N, D), jnp.float32),
        ],
    )(feat, w1cat, w2cat, whcat, bhcat, wself, bself)


def _stage_b_kernel(src_hbm, dst_hbm, tb1_hbm, tb2_hbm, out_hbm,
                    sidx, didx, abuf, bbuf, obuf, sema0, semb0, sema1, semb1):
    wid = lax.axis_index("s") * NC + lax.axis_index("c")
    base = wid * EPW
    gsems = ((sema0, semb0), (sema1, semb1))

    def issue(j, t):
        off = base + j * KB
        pltpu.sync_copy(src_hbm.at[pl.ds(off, KB)], sidx.at[t])
        pltpu.sync_copy(dst_hbm.at[pl.ds(off, KB)], didx.at[t])
        pltpu.async_copy(tb1_hbm.at[sidx.at[t]], abuf.at[t], gsems[t][0])
        pltpu.async_copy(tb2_hbm.at[didx.at[t]], bbuf.at[t], gsems[t][1])

    def wait_slot(t):
        pltpu.make_async_copy(tb1_hbm.at[sidx.at[t]], abuf.at[t],
                              gsems[t][0]).wait()
        pltpu.make_async_copy(tb2_hbm.at[didx.at[t]], bbuf.at[t],
                              gsems[t][1]).wait()

    def compute_out(j, t):
        def row(k, c2):
            avs = [abuf[t, k, pl.ds(r * L, L)] for r in range(16)]
            bvs = [bbuf[t, k, pl.ds(r * L, L)] for r in range(16)]
            for r in range(16):
                obuf[k, pl.ds(r * L, L)] = (avs[r] * bvs[r] if r < 8
                                            else avs[r] + bvs[r])
            return c2
        lax.fori_loop(0, KB, row, 0, unroll=2)
        pltpu.sync_copy(obuf, out_hbm.at[pl.ds(base + j * KB, KB)])

    issue(0, 0)

    @pl.loop(0, NB_CH - 1, step=2)
    def chunk_pair(j):
        wait_slot(0)
        issue(j + 1, 1)
        compute_out(j, 0)
        wait_slot(1)

        @pl.when(j + 2 < NB_CH)
        def _():
            issue(j + 2, 0)
        compute_out(j + 1, 1)

    wait_slot(0)
    compute_out(NB_CH - 1, 0)


def _stage_b(src, dst, tb1, tb2):
    mesh = plsc.VectorSubcoreMesh(core_axis_name="c", subcore_axis_name="s")
    f = pl.kernel(
        _stage_b_kernel,
        out_type=jax.ShapeDtypeStruct((E, 2 * D), jnp.float32),
        mesh=mesh,
        compiler_params=pltpu.CompilerParams(needs_layout_passes=False),
        scratch_types=[
            pltpu.VMEM((2, KB), jnp.int32),
            pltpu.VMEM((2, KB), jnp.int32),
            pltpu.VMEM((2, KB, 2 * D), jnp.float32),
            pltpu.VMEM((2, KB, 2 * D), jnp.float32),
            pltpu.VMEM((KB, 2 * D), jnp.float32),
            pltpu.SemaphoreType.DMA,
            pltpu.SemaphoreType.DMA,
            pltpu.SemaphoreType.DMA,
            pltpu.SemaphoreType.DMA,
        ],
    )
    return f(src, dst, tb1, tb2)


def _stage_c_body(ab_ref, wm_ref, btot_ref, wo_ref, bo_ref, out_ref):
    ab = ab_ref[...]
    pre = (jnp.dot(ab[:, :D], wm_ref[...], preferred_element_type=jnp.float32)
           + ab[:, D:] + btot_ref[...])
    x2 = _gelu(pre)
    y = jnp.dot(x2, wo_ref[...], preferred_element_type=jnp.float32) + bo_ref[...]
    out_ref[...] = jnp.where(y > 0, y, 0.2 * y)


def _stage_c(ab, wm, btot, wout2, bout2):
    blk = 3200
    grid = (E // blk,)
    return pl.pallas_call(
        _stage_c_body,
        grid=grid,
        in_specs=[
            pl.BlockSpec((blk, 2 * D), lambda i: (i, 0)),
            pl.BlockSpec((D, D), lambda i: (0, 0)),
            pl.BlockSpec((1, D), lambda i: (0, 0)),
            pl.BlockSpec((D, 1), lambda i: (0, 0)),
            pl.BlockSpec((1, 1), lambda i: (0, 0)),
        ],
        out_specs=pl.BlockSpec((blk, 1), lambda i: (i, 0)),
        out_shape=jax.ShapeDtypeStruct((E, 1), jnp.float32),
    )(ab, wm, btot, wout2, bout2)


NCH = E // CS


def _stage_d_kernel(src_hbm, dst_hbm, e_hbm, th_hbm,
                    maxout, sumout, degout,
                    maxacc, sumacc, degacc,
                    sbuf0, dbuf0, ebuf0, sbuf1, dbuf1, ebuf1,
                    wl_src, wl_row, wl_e, gbuf, csem, gsem):
    sid = lax.axis_index("s")
    wid = lax.axis_index("c") * NS + sid
    lo = wid * NPW
    iota = lax.iota(jnp.int32, L)

    zf = jnp.zeros((L,), jnp.float32)
    zi = jnp.zeros((L,), jnp.int32)
    ninf = jnp.full((L,), -1e30, jnp.float32)

    def init_rows(r, c):
        for cc in range(8):
            maxacc[r, pl.ds(cc * L, L)] = ninf
            sumacc[r, pl.ds(cc * L, L)] = zf
        return c
    lax.fori_loop(0, NPW, init_rows, 0, unroll=False)

    def init_deg(g, c):
        degacc[pl.ds(g * L, L)] = zf
        return c
    lax.fori_loop(0, NPW // L, init_deg, 0, unroll=False)

    def init_wl(g, c):
        wl_src[pl.ds(g * L, L)] = zi
        return c
    lax.fori_loop(0, WLCAP // L, init_wl, 0, unroll=False)

    bufs0 = (sbuf0, dbuf0, ebuf0)
    bufs1 = (sbuf1, dbuf1, ebuf1)

    def issue_chunk(ch, bufs):
        off = ch * CS
        pltpu.async_copy(src_hbm.at[pl.ds(off, CS)], bufs[0], csem)
        pltpu.async_copy(dst_hbm.at[pl.ds(off, CS)], bufs[1], csem)
        pltpu.async_copy(e_hbm.at[pl.ds(off, CS)], bufs[2], csem)

    def wait_chunk(bufs):
        pltpu.make_async_copy(src_hbm.at[pl.ds(0, CS)], bufs[0], csem).wait()
        pltpu.make_async_copy(src_hbm.at[pl.ds(0, CS)], bufs[1], csem).wait()
        pltpu.make_async_copy(e_hbm.at[pl.ds(0, CS)], bufs[2], csem).wait()

    colvs = [jnp.full((L,), cc * L, jnp.int32) + iota for cc in range(8)]

    def edge_work(wl_base, j):
        jv = jnp.full((L,), wl_base + j, jnp.int32)
        rowv = plsc.load_gather(wl_row, [jv])
        evv = plsc.load_gather(wl_e, [jv])
        olds = [plsc.load_gather(maxacc, [rowv, colvs[cc]]) for cc in range(8)]
        gvs = [gbuf[j, pl.ds(cc * L, L)] for cc in range(8)]
        g2s = [gbuf[j, pl.ds(D + cc * L, L)] for cc in range(8)]
        for cc in range(8):
            plsc.store_scatter(maxacc, [rowv, colvs[cc]],
                               jnp.maximum(olds[cc], evv * gvs[cc]))
        for cc in range(8):
            plsc.addupdate_scatter(sumacc, [rowv, colvs[cc]], evv * g2s[cc])

    def do_chunk(bufs, carry):
        sb, db, eb = bufs

        def scang(g, wpv):
            dv = db[pl.ds(g * L, L)]
            m = (dv >= lo) & (dv < lo + NPW)
            sv = sb[pl.ds(g * L, L)]
            ev = eb[pl.ds(g * L, L)]
            rv = dv - lo
            cum = plsc.cumsum(jnp.where(m, 1, 0))
            cntv = plsc.all_reduce_population_count(m)
            posv = wpv + cum - 1
            plsc.store_scatter(wl_src, [posv], sv, mask=m)
            plsc.store_scatter(wl_row, [posv], rv, mask=m)
            plsc.store_scatter(wl_e, [posv], ev, mask=m)
            plsc.addupdate_scatter(degacc, [rv],
                                   jnp.ones((L,), jnp.float32), mask=m)
            return wpv + cntv

        totalv = lax.fori_loop(0, CS // L, scang, carry, unroll=2)
        total = totalv[0]
        nsub = total // G

        def sub(q, c):
            pltpu.async_copy(th_hbm.at[wl_src.at[pl.ds(q * G, G)]],
                             gbuf, gsem).wait()

            def edge(j, c2):
                edge_work(q * G, j)
                return c2
            lax.fori_loop(0, G, edge, 0, unroll=2)
            return c
        lax.fori_loop(0, nsub, sub, 0, unroll=False)

        rem = total - nsub * G
        s0 = nsub * G
        for t in range(G // L):
            o = t * L
            a = wl_src[pl.ds(s0 + o, L)]
            b = wl_row[pl.ds(s0 + o, L)]
            cvals = wl_e[pl.ds(s0 + o, L)]
            wl_src[pl.ds(o, L)] = a
            wl_row[pl.ds(o, L)] = b
            wl_e[pl.ds(o, L)] = cvals
        return totalv - nsub * G

    issue_chunk(0, bufs0)

    def chunk_pair(ch, carry_in):
        wait_chunk(bufs0)
        issue_chunk(ch + 1, bufs1)
        c1 = do_chunk(bufs0, carry_in)
        wait_chunk(bufs1)

        @pl.when(ch + 2 < NCH)
        def _():
            issue_chunk(ch + 2, bufs0)
        return do_chunk(bufs1, c1)

    carryv = pl.loop(0, NCH, step=2,
                     init_carry=jnp.zeros((L,), jnp.int32))(chunk_pair)
    carry = carryv[0]

    pltpu.async_copy(th_hbm.at[wl_src.at[pl.ds(0, G)]], gbuf, gsem).wait()

    def edge_fin(j, c):
        edge_work(0, j)
        return c
    lax.fori_loop(0, carry, edge_fin, 0, unroll=False)

    pltpu.sync_copy(maxacc, maxout.at[pl.ds(lo, NPW)])
    pltpu.sync_copy(sumacc, sumout.at[pl.ds(lo, NPW)])
    pltpu.sync_copy(degacc, degout.at[pl.ds(lo, NPW)])


def _stage_d(src, dst, e, th):
    mesh = plsc.VectorSubcoreMesh(core_axis_name="c", subcore_axis_name="s")
    f = pl.kernel(
        _stage_d_kernel,
        out_type=[
            jax.ShapeDtypeStruct((NPAD, D), jnp.float32),
            jax.ShapeDtypeStruct((NPAD, D), jnp.float32),
            jax.ShapeDtypeStruct((NPAD,), jnp.float32),
        ],
        mesh=mesh,
        compiler_params=pltpu.CompilerParams(needs_layout_passes=False),
        scratch_types=[
            pltpu.VMEM((NPW, D), jnp.float32),
            pltpu.VMEM((NPW, D), jnp.float32),
            pltpu.VMEM((NPW,), jnp.float32),
            pltpu.VMEM((CS,), jnp.int32),
            pltpu.VMEM((CS,), jnp.int32),
            pltpu.VMEM((CS,), jnp.float32),
            pltpu.VMEM((CS,), jnp.int32),
            pltpu.VMEM((CS,), jnp.int32),
            pltpu.VMEM((CS,), jnp.float32),
            pltpu.VMEM((WLCAP,), jnp.int32),
            pltpu.VMEM((WLCAP,), jnp.int32),
            pltpu.VMEM((WLCAP,), jnp.float32),
            pltpu.VMEM((G, 2 * D), jnp.float32),
            pltpu.SemaphoreType.DMA,
            pltpu.SemaphoreType.DMA,
        ],
    )
    return f(src, dst, e, th)


def _stage_e_body(mx_ref, sm_ref, deg_ref, s_ref, wn_ref, bn_ref,
                  wn2_ref, bn2_ref, wm1_ref, bm1_ref, wm2_ref, bm2_ref,
                  out_ref):
    dg = deg_ref[...]
    neigh = jnp.where(dg > 0, mx_ref[...], 0.0)
    neigh2 = sm_ref[...] / jnp.maximum(dg, 1.0)
    rst = (s_ref[...]
           + jnp.dot(neigh, wn_ref[...], preferred_element_type=jnp.float32) + bn_ref[...]
           + jnp.dot(neigh2, wn2_ref[...], preferred_element_type=jnp.float32) + bn2_ref[...])
    rst = rst + jnp.dot(_gelu(rst), wm1_ref[...], preferred_element_type=jnp.float32) + bm1_ref[...]
    rst = rst + jnp.dot(_gelu(rst), wm2_ref[...], preferred_element_type=jnp.float32) + bm2_ref[...]
    out_ref[...] = rst


def _stage_e(mx, sm, deg, s, wn, bn, wn2, bn2, wm1, bm1, wm2, bm2):
    blk = 2000
    grid = (N // blk,)
    wfull = lambda: pl.BlockSpec((D, D), lambda i: (0, 0))
    bfull = lambda: pl.BlockSpec((1, D), lambda i: (0, 0))
    return pl.pallas_call(
        _stage_e_body,
        grid=grid,
        in_specs=[
            pl.BlockSpec((blk, D), lambda i: (i, 0)),
            pl.BlockSpec((blk, D), lambda i: (i, 0)),
            pl.BlockSpec((blk, 1), lambda i: (i, 0)),
            pl.BlockSpec((blk, D), lambda i: (i, 0)),
            wfull(), bfull(), wfull(), bfull(),
            wfull(), bfull(), wfull(), bfull(),
        ],
        out_specs=pl.BlockSpec((blk, D), lambda i: (i, 0)),
        out_shape=jax.ShapeDtypeStruct((N, D), jnp.float32),
    )(mx, sm, deg, s, wn, bn, wn2, bn2, wm1, bm1, wm2, bm2)


_DBG_B = False
_DBG_D = False

def kernel(feat, edge_index, params):
    wsrc, bsrc = _eff_w(params["atten_src"])
    wdst, bdst = _eff_w(params["atten_dst"])
    wsub, bsub = _eff_w(params["atten_sub"])
    wmul, bmul = _eff_w(params["atten_mul"])
    wout, bout = _eff_w(params["atten_out"])
    wp, bp = _eff_w(params["fc_pool"])
    wp2, bp2 = _eff_w(params["fc_pool2"])
    wself, bself = _eff_w(params["fc_self"])
    wn, bn = _eff_w(params["fc_neigh"])
    wn2, bn2 = _eff_w(params["fc_neigh2"])
    btot = (bsrc + bdst + bsub + bmul)[None, :]

    eye = jnp.eye(D, dtype=jnp.float32)
    w1cat = jnp.concatenate([eye, (wsrc + wsub).T], axis=1)
    w2cat = jnp.concatenate([eye, (wdst - wsub).T], axis=1)
    whcat = jnp.concatenate([wp.T, wp2.T], axis=1)
    bhcat = jnp.concatenate([bp, bp2])[None, :]

    tb1, tb2, th, s = _stage_a(feat, w1cat, w2cat, whcat, bhcat,
                               wself.T, bself[None, :])

    src = edge_index[0]
    dst = edge_index[1]
    if _DBG_B:
        ab = jnp.concatenate([tb1[src, :D] * tb2[dst, :D],
                              tb1[src, D:] + tb2[dst, D:]], axis=1)
    else:
        ab = _stage_b(src, dst, tb1, tb2)

    e2 = _stage_c(ab, wmul.T, btot, wout.T, bout[None, None, 0])
    e = e2.reshape(E)

    if _DBG_D:
        mx = jax.ops.segment_max(e[:, None] * th[src, :D], dst, num_segments=NPAD)
        mx = jnp.where(jnp.isfinite(mx), mx, -1e30)
        sm = jax.ops.segment_sum(e[:, None] * th[src, D:], dst, num_segments=NPAD)
        deg = jax.ops.segment_sum(jnp.ones((E,), jnp.float32), dst, num_segments=NPAD)
    else:
        mx, sm, deg = _stage_d(src, dst, e, th)
    mx = mx[:N]
    sm = sm[:N]
    deg = deg[:N].reshape(N, 1)

    wmlp1, bmlp1 = _eff_w(params["out_mlp"][0])
    wmlp2, bmlp2 = _eff_w(params["out_mlp"][1])
    rst = _stage_e(mx, sm, deg, s, wn.T, bn[None, :], wn2.T, bn2[None, :],
                   wmlp1.T, bmlp1[None, :], wmlp2.T, bmlp2[None, :])
    return rst


# speedup vs baseline: 1.9802x; 1.0106x over previous
import functools

import numpy as np

import jax
import jax.numpy as jnp
from jax import lax
from jax.experimental import pallas as pl
from jax.experimental.pallas import tpu as pltpu
from jax.experimental.pallas import tpu_sc as plsc

N = 10000
E = 320000
D = 128

NC = 2
NS = 16
NW = NC * NS
L = 16

NPW = 320
NPAD = NW * NPW
EPW = E // NW
KB = 80
NB_CH = EPW // KB
CS = 2000
ND_CH = EPW // CS
G = 48
WLCAP = CS + G


def _gelu(x):
    return x * 0.5 * (1.0 + lax.erf(x * 0.7071067811865476))


def _eff_w(p):
    V, g, b = p["v"], p["g"], p["b"]
    norm = jnp.sqrt(jnp.sum(V * V, axis=1, keepdims=True))
    return (g * V / norm), b


def _stage_a_body(feat_ref, w1_ref, w2_ref, wh_ref, bh_ref, ws_ref, bs_ref,
                  tb1_ref, tb2_ref, th_ref, s_ref):
    f = feat_ref[...]
    tb1_ref[...] = jnp.dot(f, w1_ref[...], preferred_element_type=jnp.float32)
    tb2_ref[...] = jnp.dot(f, w2_ref[...], preferred_element_type=jnp.float32)
    th_ref[...] = _gelu(jnp.dot(f, wh_ref[...], preferred_element_type=jnp.float32)
                        + bh_ref[...])
    s_ref[...] = jnp.dot(f, ws_ref[...], preferred_element_type=jnp.float32) + bs_ref[...]


def _stage_a(feat, w1cat, w2cat, whcat, bhcat, wself, bself):
    blk = 2000
    grid = (N // blk,)
    full = lambda shape: pl.BlockSpec(shape, lambda i: (0, 0))
    return pl.pallas_call(
        _stage_a_body,
        grid=grid,
        in_specs=[
            pl.BlockSpec((blk, D), lambda i: (i, 0)),
            full((D, 2 * D)), full((D, 2 * D)), full((D, 2 * D)),
            full((1, 2 * D)), full((D, D)), full((1, D)),
        ],
        out_specs=[
            pl.BlockSpec((blk, 2 * D), lambda i: (i, 0)),
            pl.BlockSpec((blk, 2 * D), lambda i: (i, 0)),
            pl.BlockSpec((blk, 2 * D), lambda i: (i, 0)),
            pl.BlockSpec((blk, D), lambda i: (i, 0)),
        ],
        out_shape=[
            jax.ShapeDtypeStruct((N, 2 * D), jnp.float32),
            jax.ShapeDtypeStruct((N, 2 * D), jnp.float32),
            jax.ShapeDtypeStruct((N, 2 * D), jnp.float32),
            jax.ShapeDtypeStruct((N, D), jnp.float32),
        ],
    )(feat, w1cat, w2cat, whcat, bhcat, wself, bself)


def _stage_b_kernel(src_hbm, dst_hbm, tb1_hbm, tb2_hbm, out_hbm,
                    sidx, didx, abuf, bbuf, obuf, sema0, semb0, sema1, semb1):
    wid = lax.axis_index("s") * NC + lax.axis_index("c")
    base = wid * EPW
    gsems = ((sema0, semb0), (sema1, semb1))

    def issue(j, t):
        off = base + j * KB
        pltpu.sync_copy(src_hbm.at[pl.ds(off, KB)], sidx.at[t])
        pltpu.sync_copy(dst_hbm.at[pl.ds(off, KB)], didx.at[t])
        pltpu.async_copy(tb1_hbm.at[sidx.at[t]], abuf.at[t], gsems[t][0])
        pltpu.async_copy(tb2_hbm.at[didx.at[t]], bbuf.at[t], gsems[t][1])

    def wait_slot(t):
        pltpu.make_async_copy(tb1_hbm.at[sidx.at[t]], abuf.at[t],
                              gsems[t][0]).wait()
        pltpu.make_async_copy(tb2_hbm.at[didx.at[t]], bbuf.at[t],
                              gsems[t][1]).wait()

    def compute_out(j, t):
        def row(k, c2):
            avs = [abuf[t, k, pl.ds(r * L, L)] for r in range(16)]
            bvs = [bbuf[t, k, pl.ds(r * L, L)] for r in range(16)]
            for r in range(16):
                obuf[k, pl.ds(r * L, L)] = (avs[r] * bvs[r] if r < 8
                                            else avs[r] + bvs[r])
            return c2
        lax.fori_loop(0, KB, row, 0, unroll=2)
        pltpu.sync_copy(obuf, out_hbm.at[pl.ds(base + j * KB, KB)])

    issue(0, 0)

    @pl.loop(0, NB_CH - 1, step=2)
    def chunk_pair(j):
        wait_slot(0)
        issue(j + 1, 1)
        compute_out(j, 0)
        wait_slot(1)

        @pl.when(j + 2 < NB_CH)
        def _():
            issue(j + 2, 0)
        compute_out(j + 1, 1)

    wait_slot(0)
    compute_out(NB_CH - 1, 0)


def _stage_b(src, dst, tb1, tb2):
    mesh = plsc.VectorSubcoreMesh(core_axis_name="c", subcore_axis_name="s")
    f = pl.kernel(
        _stage_b_kernel,
        out_type=jax.ShapeDtypeStruct((E, 2 * D), jnp.float32),
        mesh=mesh,
        compiler_params=pltpu.CompilerParams(needs_layout_passes=False),
        scratch_types=[
            pltpu.VMEM((2, KB), jnp.int32),
            pltpu.VMEM((2, KB), jnp.int32),
            pltpu.VMEM((2, KB, 2 * D), jnp.float32),
            pltpu.VMEM((2, KB, 2 * D), jnp.float32),
            pltpu.VMEM((KB, 2 * D), jnp.float32),
            pltpu.SemaphoreType.DMA,
            pltpu.SemaphoreType.DMA,
            pltpu.SemaphoreType.DMA,
            pltpu.SemaphoreType.DMA,
        ],
    )
    return f(src, dst, tb1, tb2)


def _stage_c_body(ab_ref, wm_ref, btot_ref, wo_ref, bo_ref, out_ref):
    ab = ab_ref[...]
    pre = (jnp.dot(ab[:, :D], wm_ref[...], preferred_element_type=jnp.float32)
           + ab[:, D:] + btot_ref[...])
    x2 = _gelu(pre)
    y = jnp.dot(x2, wo_ref[...], preferred_element_type=jnp.float32) + bo_ref[...]
    out_ref[...] = jnp.where(y > 0, y, 0.2 * y)


def _stage_c(ab, wm, btot, wout2, bout2):
    blk = 3200
    grid = (E // blk,)
    return pl.pallas_call(
        _stage_c_body,
        grid=grid,
        in_specs=[
            pl.BlockSpec((blk, 2 * D), lambda i: (i, 0)),
            pl.BlockSpec((D, D), lambda i: (0, 0)),
            pl.BlockSpec((1, D), lambda i: (0, 0)),
            pl.BlockSpec((D, 1), lambda i: (0, 0)),
            pl.BlockSpec((1, 1), lambda i: (0, 0)),
        ],
        out_specs=pl.BlockSpec((blk, 1), lambda i: (i, 0)),
        out_shape=jax.ShapeDtypeStruct((E, 1), jnp.float32),
    )(ab, wm, btot, wout2, bout2)


NCH = E // CS


def _stage_d_kernel(src_hbm, dst_hbm, e_hbm, th_hbm,
                    maxout, sumout, degout,
                    maxacc, sumacc, degacc,
                    sbuf0, dbuf0, ebuf0, sbuf1, dbuf1, ebuf1,
                    wl_src, wl_row, wl_e, gbuf0, gbuf1, csem, gsem):
    sid = lax.axis_index("s")
    wid = lax.axis_index("c") * NS + sid
    lo = wid * NPW
    iota = lax.iota(jnp.int32, L)

    zf = jnp.zeros((L,), jnp.float32)
    zi = jnp.zeros((L,), jnp.int32)
    ninf = jnp.full((L,), -1e30, jnp.float32)

    def init_rows(r, c):
        for cc in range(8):
            maxacc[r, pl.ds(cc * L, L)] = ninf
            sumacc[r, pl.ds(cc * L, L)] = zf
        return c
    lax.fori_loop(0, NPW, init_rows, 0, unroll=False)

    def init_deg(g, c):
        degacc[pl.ds(g * L, L)] = zf
        return c
    lax.fori_loop(0, NPW // L, init_deg, 0, unroll=False)

    def init_wl(g, c):
        wl_src[pl.ds(g * L, L)] = zi
        return c
    lax.fori_loop(0, WLCAP // L, init_wl, 0, unroll=False)

    bufs0 = (sbuf0, dbuf0, ebuf0)
    bufs1 = (sbuf1, dbuf1, ebuf1)

    def issue_chunk(ch, bufs):
        off = ch * CS
        pltpu.async_copy(src_hbm.at[pl.ds(off, CS)], bufs[0], csem)
        pltpu.async_copy(dst_hbm.at[pl.ds(off, CS)], bufs[1], csem)
        pltpu.async_copy(e_hbm.at[pl.ds(off, CS)], bufs[2], csem)

    def wait_chunk(bufs):
        pltpu.make_async_copy(src_hbm.at[pl.ds(0, CS)], bufs[0], csem).wait()
        pltpu.make_async_copy(src_hbm.at[pl.ds(0, CS)], bufs[1], csem).wait()
        pltpu.make_async_copy(e_hbm.at[pl.ds(0, CS)], bufs[2], csem).wait()

    colvs = [jnp.full((L,), cc * L, jnp.int32) + iota for cc in range(8)]

    def edge_work(wl_base, j, gb):
        jv = jnp.full((L,), wl_base + j, jnp.int32)
        rowv = plsc.load_gather(wl_row, [jv])
        evv = plsc.load_gather(wl_e, [jv])
        olds = [plsc.load_gather(maxacc, [rowv, colvs[cc]]) for cc in range(8)]
        gvs = [gb[j, pl.ds(cc * L, L)] for cc in range(8)]
        g2s = [gb[j, pl.ds(D + cc * L, L)] for cc in range(8)]
        for cc in range(8):
            plsc.store_scatter(maxacc, [rowv, colvs[cc]],
                               jnp.maximum(olds[cc], evv * gvs[cc]))
        for cc in range(8):
            plsc.addupdate_scatter(sumacc, [rowv, colvs[cc]], evv * g2s[cc])

    def do_chunk(bufs, carry):
        sb, db, eb = bufs

        def scang(g, wpv):
            dv = db[pl.ds(g * L, L)]
            m = (dv >= lo) & (dv < lo + NPW)
            sv = sb[pl.ds(g * L, L)]
            ev = eb[pl.ds(g * L, L)]
            rv = dv - lo
            cum = plsc.cumsum(jnp.where(m, 1, 0))
            cntv = plsc.all_reduce_population_count(m)
            posv = wpv + cum - 1
            plsc.store_scatter(wl_src, [posv], sv, mask=m)
            plsc.store_scatter(wl_row, [posv], rv, mask=m)
            plsc.store_scatter(wl_e, [posv], ev, mask=m)
            plsc.addupdate_scatter(degacc, [rv],
                                   jnp.ones((L,), jnp.float32), mask=m)
            return wpv + cntv

        totalv = lax.fori_loop(0, CS // L, scang, carry, unroll=4)
        total = totalv[0]
        nsub = total // G

        def issue_gather(q, gb):
            pltpu.async_copy(th_hbm.at[wl_src.at[pl.ds(q * G, G)]], gb, gsem)

        def wait_gather(gb):
            pltpu.make_async_copy(th_hbm.at[wl_src.at[pl.ds(0, G)]], gb,
                                  gsem).wait()

        def run_edges(q, gb):
            def edge(j, c2):
                edge_work(q * G, j, gb)
                return c2
            lax.fori_loop(0, G, edge, 0, unroll=2)

        @pl.when(nsub > 0)
        def _():
            issue_gather(0, gbuf0)

        def subpair(h, c):
            q = h * 2
            wait_gather(gbuf0)

            @pl.when(q + 1 < nsub)
            def _():
                issue_gather(q + 1, gbuf1)
            run_edges(q, gbuf0)

            @pl.when(q + 1 < nsub)
            def _():
                wait_gather(gbuf1)

                @pl.when(q + 2 < nsub)
                def _():
                    issue_gather(q + 2, gbuf0)
                run_edges(q + 1, gbuf1)
            return c
        lax.fori_loop(0, (nsub + 1) // 2, subpair, 0, unroll=False)

        rem = total - nsub * G
        s0 = nsub * G
        for t in range(G // L):
            o = t * L
            a = wl_src[pl.ds(s0 + o, L)]
            b = wl_row[pl.ds(s0 + o, L)]
            cvals = wl_e[pl.ds(s0 + o, L)]
            wl_src[pl.ds(o, L)] = a
            wl_row[pl.ds(o, L)] = b
            wl_e[pl.ds(o, L)] = cvals
        return totalv - nsub * G

    issue_chunk(0, bufs0)

    def chunk_pair(ch, carry_in):
        wait_chunk(bufs0)
        issue_chunk(ch + 1, bufs1)
        c1 = do_chunk(bufs0, carry_in)
        wait_chunk(bufs1)

        @pl.when(ch + 2 < NCH)
        def _():
            issue_chunk(ch + 2, bufs0)
        return do_chunk(bufs1, c1)

    carryv = pl.loop(0, NCH, step=2,
                     init_carry=jnp.zeros((L,), jnp.int32))(chunk_pair)
    carry = carryv[0]

    pltpu.async_copy(th_hbm.at[wl_src.at[pl.ds(0, G)]], gbuf0, gsem).wait()

    def edge_fin(j, c):
        edge_work(0, j, gbuf0)
        return c
    lax.fori_loop(0, carry, edge_fin, 0, unroll=False)

    pltpu.sync_copy(maxacc, maxout.at[pl.ds(lo, NPW)])
    pltpu.sync_copy(sumacc, sumout.at[pl.ds(lo, NPW)])
    pltpu.sync_copy(degacc, degout.at[pl.ds(lo, NPW)])


def _stage_d(src, dst, e, th):
    mesh = plsc.VectorSubcoreMesh(core_axis_name="c", subcore_axis_name="s")
    f = pl.kernel(
        _stage_d_kernel,
        out_type=[
            jax.ShapeDtypeStruct((NPAD, D), jnp.float32),
            jax.ShapeDtypeStruct((NPAD, D), jnp.float32),
            jax.ShapeDtypeStruct((NPAD,), jnp.float32),
        ],
        mesh=mesh,
        compiler_params=pltpu.CompilerParams(needs_layout_passes=False),
        scratch_types=[
            pltpu.VMEM((NPW, D), jnp.float32),
            pltpu.VMEM((NPW, D), jnp.float32),
            pltpu.VMEM((NPW,), jnp.float32),
            pltpu.VMEM((CS,), jnp.int32),
            pltpu.VMEM((CS,), jnp.int32),
            pltpu.VMEM((CS,), jnp.float32),
            pltpu.VMEM((CS,), jnp.int32),
            pltpu.VMEM((CS,), jnp.int32),
            pltpu.VMEM((CS,), jnp.float32),
            pltpu.VMEM((WLCAP,), jnp.int32),
            pltpu.VMEM((WLCAP,), jnp.int32),
            pltpu.VMEM((WLCAP,), jnp.float32),
            pltpu.VMEM((G, 2 * D), jnp.float32),
            pltpu.VMEM((G, 2 * D), jnp.float32),
            pltpu.SemaphoreType.DMA,
            pltpu.SemaphoreType.DMA,
        ],
    )
    return f(src, dst, e, th)


def _stage_e_body(mx_ref, sm_ref, deg_ref, s_ref, wn_ref, bn_ref,
                  wn2_ref, bn2_ref, wm1_ref, bm1_ref, wm2_ref, bm2_ref,
                  out_ref):
    dg = deg_ref[...]
    neigh = jnp.where(dg > 0, mx_ref[...], 0.0)
    neigh2 = sm_ref[...] / jnp.maximum(dg, 1.0)
    rst = (s_ref[...]
           + jnp.dot(neigh, wn_ref[...], preferred_element_type=jnp.float32) + bn_ref[...]
           + jnp.dot(neigh2, wn2_ref[...], preferred_element_type=jnp.float32) + bn2_ref[...])
    rst = rst + jnp.dot(_gelu(rst), wm1_ref[...], preferred_element_type=jnp.float32) + bm1_ref[...]
    rst = rst + jnp.dot(_gelu(rst), wm2_ref[...], preferred_element_type=jnp.float32) + bm2_ref[...]
    out_ref[...] = rst


def _stage_e(mx, sm, deg, s, wn, bn, wn2, bn2, wm1, bm1, wm2, bm2):
    blk = 2000
    grid = (N // blk,)
    wfull = lambda: pl.BlockSpec((D, D), lambda i: (0, 0))
    bfull = lambda: pl.BlockSpec((1, D), lambda i: (0, 0))
    return pl.pallas_call(
        _stage_e_body,
        grid=grid,
        in_specs=[
            pl.BlockSpec((blk, D), lambda i: (i, 0)),
            pl.BlockSpec((blk, D), lambda i: (i, 0)),
            pl.BlockSpec((blk, 1), lambda i: (i, 0)),
            pl.BlockSpec((blk, D), lambda i: (i, 0)),
            wfull(), bfull(), wfull(), bfull(),
            wfull(), bfull(), wfull(), bfull(),
        ],
        out_specs=pl.BlockSpec((blk, D), lambda i: (i, 0)),
        out_shape=jax.ShapeDtypeStruct((N, D), jnp.float32),
    )(mx, sm, deg, s, wn, bn, wn2, bn2, wm1, bm1, wm2, bm2)


_DBG_B = False
_DBG_D = False

def kernel(feat, edge_index, params):
    wsrc, bsrc = _eff_w(params["atten_src"])
    wdst, bdst = _eff_w(params["atten_dst"])
    wsub, bsub = _eff_w(params["atten_sub"])
    wmul, bmul = _eff_w(params["atten_mul"])
    wout, bout = _eff_w(params["atten_out"])
    wp, bp = _eff_w(params["fc_pool"])
    wp2, bp2 = _eff_w(params["fc_pool2"])
    wself, bself = _eff_w(params["fc_self"])
    wn, bn = _eff_w(params["fc_neigh"])
    wn2, bn2 = _eff_w(params["fc_neigh2"])
    btot = (bsrc + bdst + bsub + bmul)[None, :]

    eye = jnp.eye(D, dtype=jnp.float32)
    w1cat = jnp.concatenate([eye, (wsrc + wsub).T], axis=1)
    w2cat = jnp.concatenate([eye, (wdst - wsub).T], axis=1)
    whcat = jnp.concatenate([wp.T, wp2.T], axis=1)
    bhcat = jnp.concatenate([bp, bp2])[None, :]

    tb1, tb2, th, s = _stage_a(feat, w1cat, w2cat, whcat, bhcat,
                               wself.T, bself[None, :])

    src = edge_index[0]
    dst = edge_index[1]
    if _DBG_B:
        ab = jnp.concatenate([tb1[src, :D] * tb2[dst, :D],
                              tb1[src, D:] + tb2[dst, D:]], axis=1)
    else:
        ab = _stage_b(src, dst, tb1, tb2)

    e2 = _stage_c(ab, wmul.T, btot, wout.T, bout[None, None, 0])
    e = e2.reshape(E)

    if _DBG_D:
        mx = jax.ops.segment_max(e[:, None] * th[src, :D], dst, num_segments=NPAD)
        mx = jnp.where(jnp.isfinite(mx), mx, -1e30)
        sm = jax.ops.segment_sum(e[:, None] * th[src, D:], dst, num_segments=NPAD)
        deg = jax.ops.segment_sum(jnp.ones((E,), jnp.float32), dst, num_segments=NPAD)
    else:
        mx, sm, deg = _stage_d(src, dst, e, th)
    mx = mx[:N]
    sm = sm[:N]
    deg = deg[:N].reshape(N, 1)

    wmlp1, bmlp1 = _eff_w(params["out_mlp"][0])
    wmlp2, bmlp2 = _eff_w(params["out_mlp"][1])
    rst = _stage_e(mx, sm, deg, s, wn.T, bn[None, :], wn2.T, bn2[None, :],
                   wmlp1.T, bmlp1[None, :], wmlp2.T, bmlp2[None, :])
    return rst


# speedup vs baseline: 2.2176x; 1.1199x over previous
import functools

import numpy as np

import jax
import jax.numpy as jnp
from jax import lax
from jax.experimental import pallas as pl
from jax.experimental.pallas import tpu as pltpu
from jax.experimental.pallas import tpu_sc as plsc

N = 10000
E = 320000
D = 128

NC = 2
NS = 16
NW = NC * NS
L = 16

NPW = 320
NPAD = NW * NPW
EPW = E // NW
KB = 80
NB_CH = EPW // KB
CS = 2000
ND_CH = EPW // CS
G = 48
WLCAP = CS + G


def _gelu(x):
    return x * 0.5 * (1.0 + lax.erf(x * 0.7071067811865476))


def _eff_w(p):
    V, g, b = p["v"], p["g"], p["b"]
    norm = jnp.sqrt(jnp.sum(V * V, axis=1, keepdims=True))
    return (g * V / norm), b


def _stage_a_body(feat_ref, w1_ref, w2_ref, wh_ref, bh_ref, ws_ref, bs_ref,
                  tb1_ref, tb2_ref, th_ref, s_ref):
    f = feat_ref[...]
    tb1_ref[...] = jnp.dot(f, w1_ref[...], preferred_element_type=jnp.float32)
    tb2_ref[...] = jnp.dot(f, w2_ref[...], preferred_element_type=jnp.float32)
    th_ref[...] = _gelu(jnp.dot(f, wh_ref[...], preferred_element_type=jnp.float32)
                        + bh_ref[...])
    s_ref[...] = jnp.dot(f, ws_ref[...], preferred_element_type=jnp.float32) + bs_ref[...]


def _stage_a(feat, w1cat, w2cat, whcat, bhcat, wself, bself):
    blk = 2000
    grid = (N // blk,)
    full = lambda shape: pl.BlockSpec(shape, lambda i: (0, 0))
    return pl.pallas_call(
        _stage_a_body,
        grid=grid,
        in_specs=[
            pl.BlockSpec((blk, D), lambda i: (i, 0)),
            full((D, 2 * D)), full((D, 2 * D)), full((D, 2 * D)),
            full((1, 2 * D)), full((D, D)), full((1, D)),
        ],
        out_specs=[
            pl.BlockSpec((blk, 2 * D), lambda i: (i, 0)),
            pl.BlockSpec((blk, 2 * D), lambda i: (i, 0)),
            pl.BlockSpec((blk, 2 * D), lambda i: (i, 0)),
            pl.BlockSpec((blk, D), lambda i: (i, 0)),
        ],
        out_shape=[
            jax.ShapeDtypeStruct((N, 2 * D), jnp.float32),
            jax.ShapeDtypeStruct((N, 2 * D), jnp.float32),
            jax.ShapeDtypeStruct((N, 2 * D), jnp.float32),
            jax.ShapeDtypeStruct((N, D), jnp.float32),
        ],
    )(feat, w1cat, w2cat, whcat, bhcat, wself, bself)


def _stage_b_kernel(src_hbm, dst_hbm, tb1_hbm, tb2_hbm, out_hbm,
                    sidx, didx, abuf, bbuf, obuf, sema0, semb0, sema1, semb1,
                    osem):
    wid = lax.axis_index("s") * NC + lax.axis_index("c")
    base = wid * EPW
    gsems = ((sema0, semb0), (sema1, semb1))

    pltpu.sync_copy(src_hbm.at[pl.ds(base, EPW)], sidx)
    pltpu.sync_copy(dst_hbm.at[pl.ds(base, EPW)], didx)

    def issue(j, t):
        pltpu.async_copy(tb1_hbm.at[sidx.at[pl.ds(j * KB, KB)]],
                         abuf.at[t], gsems[t][0])
        pltpu.async_copy(tb2_hbm.at[didx.at[pl.ds(j * KB, KB)]],
                         bbuf.at[t], gsems[t][1])

    def wait_slot(j, t):
        pltpu.make_async_copy(tb1_hbm.at[sidx.at[pl.ds(j * KB, KB)]],
                              abuf.at[t], gsems[t][0]).wait()
        pltpu.make_async_copy(tb2_hbm.at[didx.at[pl.ds(j * KB, KB)]],
                              bbuf.at[t], gsems[t][1]).wait()

    def compute_out(j, t):
        @pl.when(j > 0)
        def _():
            pltpu.make_async_copy(obuf, out_hbm.at[pl.ds(base, KB)],
                                  osem).wait()

        def row(k, c2):
            avs = [abuf[t, k, pl.ds(r * L, L)] for r in range(16)]
            bvs = [bbuf[t, k, pl.ds(r * L, L)] for r in range(16)]
            for r in range(16):
                obuf[k, pl.ds(r * L, L)] = (avs[r] * bvs[r] if r < 8
                                            else avs[r] + bvs[r])
            return c2
        lax.fori_loop(0, KB, row, 0, unroll=2)
        pltpu.async_copy(obuf, out_hbm.at[pl.ds(base + j * KB, KB)], osem)

    issue(0, 0)

    @pl.loop(0, NB_CH - 1, step=2)
    def chunk_pair(j):
        wait_slot(j, 0)
        issue(j + 1, 1)
        compute_out(j, 0)
        wait_slot(j + 1, 1)

        @pl.when(j + 2 < NB_CH)
        def _():
            issue(j + 2, 0)
        compute_out(j + 1, 1)

    wait_slot(NB_CH - 1, 0)
    compute_out(NB_CH - 1, 0)
    pltpu.make_async_copy(obuf, out_hbm.at[pl.ds(base, KB)], osem).wait()


def _stage_b(src, dst, tb1, tb2):
    mesh = plsc.VectorSubcoreMesh(core_axis_name="c", subcore_axis_name="s")
    f = pl.kernel(
        _stage_b_kernel,
        out_type=jax.ShapeDtypeStruct((E, 2 * D), jnp.float32),
        mesh=mesh,
        compiler_params=pltpu.CompilerParams(needs_layout_passes=False),
        scratch_types=[
            pltpu.VMEM((EPW,), jnp.int32),
            pltpu.VMEM((EPW,), jnp.int32),
            pltpu.VMEM((2, KB, 2 * D), jnp.float32),
            pltpu.VMEM((2, KB, 2 * D), jnp.float32),
            pltpu.VMEM((KB, 2 * D), jnp.float32),
            pltpu.SemaphoreType.DMA,
            pltpu.SemaphoreType.DMA,
            pltpu.SemaphoreType.DMA,
            pltpu.SemaphoreType.DMA,
            pltpu.SemaphoreType.DMA,
        ],
    )
    return f(src, dst, tb1, tb2)


def _stage_c_body(ab_ref, wm_ref, btot_ref, wo_ref, bo_ref, out_ref):
    ab = ab_ref[...]
    pre = (jnp.dot(ab[:, :D], wm_ref[...], preferred_element_type=jnp.float32)
           + ab[:, D:] + btot_ref[...])
    x2 = _gelu(pre)
    y = jnp.dot(x2, wo_ref[...], preferred_element_type=jnp.float32) + bo_ref[...]
    out_ref[...] = jnp.where(y > 0, y, 0.2 * y)


def _stage_c(ab, wm, btot, wout2, bout2):
    blk = 3200
    grid = (E // blk,)
    return pl.pallas_call(
        _stage_c_body,
        grid=grid,
        in_specs=[
            pl.BlockSpec((blk, 2 * D), lambda i: (i, 0)),
            pl.BlockSpec((D, D), lambda i: (0, 0)),
            pl.BlockSpec((1, D), lambda i: (0, 0)),
            pl.BlockSpec((D, 1), lambda i: (0, 0)),
            pl.BlockSpec((1, 1), lambda i: (0, 0)),
        ],
        out_specs=pl.BlockSpec((blk, 1), lambda i: (i, 0)),
        out_shape=jax.ShapeDtypeStruct((E, 1), jnp.float32),
    )(ab, wm, btot, wout2, bout2)


NCH = E // CS


def _stage_d_kernel(src_hbm, dst_hbm, e_hbm, th_hbm,
                    maxout, sumout, degout,
                    maxacc, sumacc, degacc,
                    sbuf0, dbuf0, ebuf0, sbuf1, dbuf1, ebuf1,
                    wl_src, wl_row, wl_e, gbuf0, gbuf1, csem, gsem):
    sid = lax.axis_index("s")
    wid = lax.axis_index("c") * NS + sid
    lo = wid * NPW
    iota = lax.iota(jnp.int32, L)

    zf = jnp.zeros((L,), jnp.float32)
    zi = jnp.zeros((L,), jnp.int32)
    ninf = jnp.full((L,), -1e30, jnp.float32)

    def init_rows(r, c):
        for cc in range(8):
            maxacc[r, pl.ds(cc * L, L)] = ninf
            sumacc[r, pl.ds(cc * L, L)] = zf
        return c
    lax.fori_loop(0, NPW, init_rows, 0, unroll=False)

    def init_deg(g, c):
        degacc[pl.ds(g * L, L)] = zf
        return c
    lax.fori_loop(0, NPW // L, init_deg, 0, unroll=False)

    def init_wl(g, c):
        wl_src[pl.ds(g * L, L)] = zi
        return c
    lax.fori_loop(0, WLCAP // L, init_wl, 0, unroll=False)

    bufs0 = (sbuf0, dbuf0, ebuf0)
    bufs1 = (sbuf1, dbuf1, ebuf1)

    def issue_chunk(ch, bufs):
        off = ch * CS
        pltpu.async_copy(src_hbm.at[pl.ds(off, CS)], bufs[0], csem)
        pltpu.async_copy(dst_hbm.at[pl.ds(off, CS)], bufs[1], csem)
        pltpu.async_copy(e_hbm.at[pl.ds(off, CS)], bufs[2], csem)

    def wait_chunk(bufs):
        pltpu.make_async_copy(src_hbm.at[pl.ds(0, CS)], bufs[0], csem).wait()
        pltpu.make_async_copy(src_hbm.at[pl.ds(0, CS)], bufs[1], csem).wait()
        pltpu.make_async_copy(e_hbm.at[pl.ds(0, CS)], bufs[2], csem).wait()

    colvs = [jnp.full((L,), cc * L, jnp.int32) + iota for cc in range(8)]

    def edge_work(wl_base, j, gb):
        jv = jnp.full((L,), wl_base + j, jnp.int32)
        rowv = plsc.load_gather(wl_row, [jv])
        evv = plsc.load_gather(wl_e, [jv])
        olds = [plsc.load_gather(maxacc, [rowv, colvs[cc]]) for cc in range(8)]
        gvs = [gb[j, pl.ds(cc * L, L)] for cc in range(8)]
        g2s = [gb[j, pl.ds(D + cc * L, L)] for cc in range(8)]
        for cc in range(8):
            plsc.store_scatter(maxacc, [rowv, colvs[cc]],
                               jnp.maximum(olds[cc], evv * gvs[cc]))
        for cc in range(8):
            plsc.addupdate_scatter(sumacc, [rowv, colvs[cc]], evv * g2s[cc])

    def do_chunk(bufs, carry):
        sb, db, eb = bufs

        def scang(g, wpv):
            dv = db[pl.ds(g * L, L)]
            m = (dv >= lo) & (dv < lo + NPW)
            sv = sb[pl.ds(g * L, L)]
            ev = eb[pl.ds(g * L, L)]
            rv = dv - lo
            cum = plsc.cumsum(jnp.where(m, 1, 0))
            cntv = plsc.all_reduce_population_count(m)
            posv = wpv + cum - 1
            plsc.store_scatter(wl_src, [posv], sv, mask=m)
            plsc.store_scatter(wl_row, [posv], rv, mask=m)
            plsc.store_scatter(wl_e, [posv], ev, mask=m)
            plsc.addupdate_scatter(degacc, [rv],
                                   jnp.ones((L,), jnp.float32), mask=m)
            return wpv + cntv

        totalv = lax.fori_loop(0, CS // L, scang, carry, unroll=4)
        total = totalv[0]
        nsub = total // G

        def issue_gather(q, gb):
            pltpu.async_copy(th_hbm.at[wl_src.at[pl.ds(q * G, G)]], gb, gsem)

        def wait_gather(gb):
            pltpu.make_async_copy(th_hbm.at[wl_src.at[pl.ds(0, G)]], gb,
                                  gsem).wait()

        def run_edges(q, gb):
            def edge(j, c2):
                edge_work(q * G, j, gb)
                return c2
            lax.fori_loop(0, G, edge, 0, unroll=2)

        @pl.when(nsub > 0)
        def _():
            issue_gather(0, gbuf0)

        def subpair(h, c):
            q = h * 2
            wait_gather(gbuf0)

            @pl.when(q + 1 < nsub)
            def _():
                issue_gather(q + 1, gbuf1)
            run_edges(q, gbuf0)

            @pl.when(q + 1 < nsub)
            def _():
                wait_gather(gbuf1)

                @pl.when(q + 2 < nsub)
                def _():
                    issue_gather(q + 2, gbuf0)
                run_edges(q + 1, gbuf1)
            return c
        lax.fori_loop(0, (nsub + 1) // 2, subpair, 0, unroll=False)

        rem = total - nsub * G
        s0 = nsub * G
        for t in range(G // L):
            o = t * L
            a = wl_src[pl.ds(s0 + o, L)]
            b = wl_row[pl.ds(s0 + o, L)]
            cvals = wl_e[pl.ds(s0 + o, L)]
            wl_src[pl.ds(o, L)] = a
            wl_row[pl.ds(o, L)] = b
            wl_e[pl.ds(o, L)] = cvals
        return totalv - nsub * G

    issue_chunk(0, bufs0)

    def chunk_pair(ch, carry_in):
        wait_chunk(bufs0)
        issue_chunk(ch + 1, bufs1)
        c1 = do_chunk(bufs0, carry_in)
        wait_chunk(bufs1)

        @pl.when(ch + 2 < NCH)
        def _():
            issue_chunk(ch + 2, bufs0)
        return do_chunk(bufs1, c1)

    carryv = pl.loop(0, NCH, step=2,
                     init_carry=jnp.zeros((L,), jnp.int32))(chunk_pair)
    carry = carryv[0]

    pltpu.async_copy(th_hbm.at[wl_src.at[pl.ds(0, G)]], gbuf0, gsem).wait()

    def edge_fin(j, c):
        edge_work(0, j, gbuf0)
        return c
    lax.fori_loop(0, carry, edge_fin, 0, unroll=False)

    pltpu.sync_copy(maxacc, maxout.at[pl.ds(lo, NPW)])
    pltpu.sync_copy(sumacc, sumout.at[pl.ds(lo, NPW)])
    pltpu.sync_copy(degacc, degout.at[pl.ds(lo, NPW)])


def _stage_d(src, dst, e, th):
    mesh = plsc.VectorSubcoreMesh(core_axis_name="c", subcore_axis_name="s")
    f = pl.kernel(
        _stage_d_kernel,
        out_type=[
            jax.ShapeDtypeStruct((NPAD, D), jnp.float32),
            jax.ShapeDtypeStruct((NPAD, D), jnp.float32),
            jax.ShapeDtypeStruct((NPAD,), jnp.float32),
        ],
        mesh=mesh,
        compiler_params=pltpu.CompilerParams(needs_layout_passes=False),
        scratch_types=[
            pltpu.VMEM((NPW, D), jnp.float32),
            pltpu.VMEM((NPW, D), jnp.float32),
            pltpu.VMEM((NPW,), jnp.float32),
            pltpu.VMEM((CS,), jnp.int32),
            pltpu.VMEM((CS,), jnp.int32),
            pltpu.VMEM((CS,), jnp.float32),
            pltpu.VMEM((CS,), jnp.int32),
            pltpu.VMEM((CS,), jnp.int32),
            pltpu.VMEM((CS,), jnp.float32),
            pltpu.VMEM((WLCAP,), jnp.int32),
            pltpu.VMEM((WLCAP,), jnp.int32),
            pltpu.VMEM((WLCAP,), jnp.float32),
            pltpu.VMEM((G, 2 * D), jnp.float32),
            pltpu.VMEM((G, 2 * D), jnp.float32),
            pltpu.SemaphoreType.DMA,
            pltpu.SemaphoreType.DMA,
        ],
    )
    return f(src, dst, e, th)


def _stage_e_body(mx_ref, sm_ref, deg_ref, s_ref, wn_ref, bn_ref,
                  wn2_ref, bn2_ref, wm1_ref, bm1_ref, wm2_ref, bm2_ref,
                  out_ref):
    dg = deg_ref[...]
    neigh = jnp.where(dg > 0, mx_ref[...], 0.0)
    neigh2 = sm_ref[...] / jnp.maximum(dg, 1.0)
    rst = (s_ref[...]
           + jnp.dot(neigh, wn_ref[...], preferred_element_type=jnp.float32) + bn_ref[...]
           + jnp.dot(neigh2, wn2_ref[...], preferred_element_type=jnp.float32) + bn2_ref[...])
    rst = rst + jnp.dot(_gelu(rst), wm1_ref[...], preferred_element_type=jnp.float32) + bm1_ref[...]
    rst = rst + jnp.dot(_gelu(rst), wm2_ref[...], preferred_element_type=jnp.float32) + bm2_ref[...]
    out_ref[...] = rst


def _stage_e(mx, sm, deg, s, wn, bn, wn2, bn2, wm1, bm1, wm2, bm2):
    blk = 2000
    grid = (N // blk,)
    wfull = lambda: pl.BlockSpec((D, D), lambda i: (0, 0))
    bfull = lambda: pl.BlockSpec((1, D), lambda i: (0, 0))
    return pl.pallas_call(
        _stage_e_body,
        grid=grid,
        in_specs=[
            pl.BlockSpec((blk, D), lambda i: (i, 0)),
            pl.BlockSpec((blk, D), lambda i: (i, 0)),
            pl.BlockSpec((blk, 1), lambda i: (i, 0)),
            pl.BlockSpec((blk, D), lambda i: (i, 0)),
            wfull(), bfull(), wfull(), bfull(),
            wfull(), bfull(), wfull(), bfull(),
        ],
        out_specs=pl.BlockSpec((blk, D), lambda i: (i, 0)),
        out_shape=jax.ShapeDtypeStruct((N, D), jnp.float32),
    )(mx, sm, deg, s, wn, bn, wn2, bn2, wm1, bm1, wm2, bm2)


_DBG_B = False
_DBG_D = False

def kernel(feat, edge_index, params):
    wsrc, bsrc = _eff_w(params["atten_src"])
    wdst, bdst = _eff_w(params["atten_dst"])
    wsub, bsub = _eff_w(params["atten_sub"])
    wmul, bmul = _eff_w(params["atten_mul"])
    wout, bout = _eff_w(params["atten_out"])
    wp, bp = _eff_w(params["fc_pool"])
    wp2, bp2 = _eff_w(params["fc_pool2"])
    wself, bself = _eff_w(params["fc_self"])
    wn, bn = _eff_w(params["fc_neigh"])
    wn2, bn2 = _eff_w(params["fc_neigh2"])
    btot = (bsrc + bdst + bsub + bmul)[None, :]

    eye = jnp.eye(D, dtype=jnp.float32)
    w1cat = jnp.concatenate([eye, (wsrc + wsub).T], axis=1)
    w2cat = jnp.concatenate([eye, (wdst - wsub).T], axis=1)
    whcat = jnp.concatenate([wp.T, wp2.T], axis=1)
    bhcat = jnp.concatenate([bp, bp2])[None, :]

    tb1, tb2, th, s = _stage_a(feat, w1cat, w2cat, whcat, bhcat,
                               wself.T, bself[None, :])

    src = edge_index[0]
    dst = edge_index[1]
    if _DBG_B:
        ab = jnp.concatenate([tb1[src, :D] * tb2[dst, :D],
                              tb1[src, D:] + tb2[dst, D:]], axis=1)
    else:
        ab = _stage_b(src, dst, tb1, tb2)

    e2 = _stage_c(ab, wmul.T, btot, wout.T, bout[None, None, 0])
    e = e2.reshape(E)

    if _DBG_D:
        mx = jax.ops.segment_max(e[:, None] * th[src, :D], dst, num_segments=NPAD)
        mx = jnp.where(jnp.isfinite(mx), mx, -1e30)
        sm = jax.ops.segment_sum(e[:, None] * th[src, D:], dst, num_segments=NPAD)
        deg = jax.ops.segment_sum(jnp.ones((E,), jnp.float32), dst, num_segments=NPAD)
    else:
        mx, sm, deg = _stage_d(src, dst, e, th)
    deg = deg.reshape(NPAD, 1)

    wmlp1, bmlp1 = _eff_w(params["out_mlp"][0])
    wmlp2, bmlp2 = _eff_w(params["out_mlp"][1])
    rst = _stage_e(mx, sm, deg, s, wn.T, bn[None, :], wn2.T, bn2[None, :],
                   wmlp1.T, bmlp1[None, :], wmlp2.T, bmlp2[None, :])
    return rst


# speedup vs baseline: 2.4383x; 1.0995x over previous
import functools

import numpy as np

import jax
import jax.numpy as jnp
from jax import lax
from jax.experimental import pallas as pl
from jax.experimental.pallas import tpu as pltpu
from jax.experimental.pallas import tpu_sc as plsc

N = 10000
E = 320000
D = 128

NC = 2
NS = 16
NW = NC * NS
L = 16

NPW = 320
NPAD = NW * NPW
EPW = E // NW
KB = 80
NB_CH = EPW // KB
CS = 2000
ND_CH = EPW // CS
G = 32
WLCAP = CS + G


def _gelu(x):
    return x * 0.5 * (1.0 + lax.erf(x * 0.7071067811865476))


def _eff_w(p):
    V, g, b = p["v"], p["g"], p["b"]
    norm = jnp.sqrt(jnp.sum(V * V, axis=1, keepdims=True))
    return (g * V / norm), b


def _stage_a_body(feat_ref, w1_ref, w2_ref, wh_ref, bh_ref, ws_ref, bs_ref,
                  tb1_ref, tb2_ref, th_ref, s_ref):
    f = feat_ref[...]
    tb1_ref[...] = jnp.dot(f, w1_ref[...], preferred_element_type=jnp.float32)
    tb2_ref[...] = jnp.dot(f, w2_ref[...], preferred_element_type=jnp.float32)
    th_ref[...] = _gelu(jnp.dot(f, wh_ref[...], preferred_element_type=jnp.float32)
                        + bh_ref[...])
    s_ref[...] = jnp.dot(f, ws_ref[...], preferred_element_type=jnp.float32) + bs_ref[...]


def _stage_a(feat, w1cat, w2cat, whcat, bhcat, wself, bself):
    blk = 2000
    grid = (N // blk,)
    full = lambda shape: pl.BlockSpec(shape, lambda i: (0, 0))
    return pl.pallas_call(
        _stage_a_body,
        grid=grid,
        in_specs=[
            pl.BlockSpec((blk, D), lambda i: (i, 0)),
            full((D, 2 * D)), full((D, 2 * D)), full((D, 2 * D)),
            full((1, 2 * D)), full((D, D)), full((1, D)),
        ],
        out_specs=[
            pl.BlockSpec((blk, 2 * D), lambda i: (i, 0)),
            pl.BlockSpec((blk, 2 * D), lambda i: (i, 0)),
            pl.BlockSpec((blk, 2 * D), lambda i: (i, 0)),
            pl.BlockSpec((blk, D), lambda i: (i, 0)),
        ],
        out_shape=[
            jax.ShapeDtypeStruct((N, 2 * D), jnp.float32),
            jax.ShapeDtypeStruct((N, 2 * D), jnp.float32),
            jax.ShapeDtypeStruct((N, 2 * D), jnp.float32),
            jax.ShapeDtypeStruct((N, D), jnp.float32),
        ],
    )(feat, w1cat, w2cat, whcat, bhcat, wself, bself)


def _stage_b_kernel(src_hbm, dst_hbm, tb1_hbm, tb2_hbm, out_hbm,
                    sidx, didx, abuf, bbuf, obuf, sema0, semb0, sema1, semb1,
                    osem):
    wid = lax.axis_index("s") * NC + lax.axis_index("c")
    base = wid * EPW
    gsems = ((sema0, semb0), (sema1, semb1))

    pltpu.sync_copy(src_hbm.at[pl.ds(base, EPW)], sidx)
    pltpu.sync_copy(dst_hbm.at[pl.ds(base, EPW)], didx)

    def issue(j, t):
        pltpu.async_copy(tb1_hbm.at[sidx.at[pl.ds(j * KB, KB)]],
                         abuf.at[t], gsems[t][0])
        pltpu.async_copy(tb2_hbm.at[didx.at[pl.ds(j * KB, KB)]],
                         bbuf.at[t], gsems[t][1])

    def wait_slot(j, t):
        pltpu.make_async_copy(tb1_hbm.at[sidx.at[pl.ds(j * KB, KB)]],
                              abuf.at[t], gsems[t][0]).wait()
        pltpu.make_async_copy(tb2_hbm.at[didx.at[pl.ds(j * KB, KB)]],
                              bbuf.at[t], gsems[t][1]).wait()

    def compute_out(j, t):
        @pl.when(j > 0)
        def _():
            pltpu.make_async_copy(obuf, out_hbm.at[pl.ds(base, KB)],
                                  osem).wait()

        def row(k, c2):
            avs = [abuf[t, k, pl.ds(r * L, L)] for r in range(16)]
            bvs = [bbuf[t, k, pl.ds(r * L, L)] for r in range(16)]
            for r in range(16):
                obuf[k, pl.ds(r * L, L)] = (avs[r] * bvs[r] if r < 8
                                            else avs[r] + bvs[r])
            return c2
        lax.fori_loop(0, KB, row, 0, unroll=2)
        pltpu.async_copy(obuf, out_hbm.at[pl.ds(base + j * KB, KB)], osem)

    issue(0, 0)

    @pl.loop(0, NB_CH - 1, step=2)
    def chunk_pair(j):
        wait_slot(j, 0)
        issue(j + 1, 1)
        compute_out(j, 0)
        wait_slot(j + 1, 1)

        @pl.when(j + 2 < NB_CH)
        def _():
            issue(j + 2, 0)
        compute_out(j + 1, 1)

    wait_slot(NB_CH - 1, 0)
    compute_out(NB_CH - 1, 0)
    pltpu.make_async_copy(obuf, out_hbm.at[pl.ds(base, KB)], osem).wait()


def _stage_b(src, dst, tb1, tb2):
    mesh = plsc.VectorSubcoreMesh(core_axis_name="c", subcore_axis_name="s")
    f = pl.kernel(
        _stage_b_kernel,
        out_type=jax.ShapeDtypeStruct((E, 2 * D), jnp.float32),
        mesh=mesh,
        compiler_params=pltpu.CompilerParams(needs_layout_passes=False),
        scratch_types=[
            pltpu.VMEM((EPW,), jnp.int32),
            pltpu.VMEM((EPW,), jnp.int32),
            pltpu.VMEM((2, KB, 2 * D), jnp.float32),
            pltpu.VMEM((2, KB, 2 * D), jnp.float32),
            pltpu.VMEM((KB, 2 * D), jnp.float32),
            pltpu.SemaphoreType.DMA,
            pltpu.SemaphoreType.DMA,
            pltpu.SemaphoreType.DMA,
            pltpu.SemaphoreType.DMA,
            pltpu.SemaphoreType.DMA,
        ],
    )
    return f(src, dst, tb1, tb2)


def _stage_c_body(ab_ref, wm_ref, btot_ref, wo_ref, bo_ref, out_ref):
    ab = ab_ref[...]
    pre = (jnp.dot(ab[:, :D], wm_ref[...], preferred_element_type=jnp.float32)
           + ab[:, D:] + btot_ref[...])
    x2 = _gelu(pre)
    y = jnp.dot(x2, wo_ref[...], preferred_element_type=jnp.float32) + bo_ref[...]
    out_ref[...] = jnp.where(y > 0, y, 0.2 * y)


def _stage_c(ab, wm, btot, wout2, bout2):
    blk = 3200
    grid = (E // blk,)
    return pl.pallas_call(
        _stage_c_body,
        grid=grid,
        in_specs=[
            pl.BlockSpec((blk, 2 * D), lambda i: (i, 0)),
            pl.BlockSpec((D, D), lambda i: (0, 0)),
            pl.BlockSpec((1, D), lambda i: (0, 0)),
            pl.BlockSpec((D, 1), lambda i: (0, 0)),
            pl.BlockSpec((1, 1), lambda i: (0, 0)),
        ],
        out_specs=pl.BlockSpec((blk, 1), lambda i: (i, 0)),
        out_shape=jax.ShapeDtypeStruct((E, 1), jnp.float32),
    )(ab, wm, btot, wout2, bout2)


NCH = E // CS


def _stage_d_kernel(src_hbm, dst_hbm, e_hbm, th_hbm,
                    maxout, sumout, degout,
                    maxacc, sumacc, degacc,
                    sbuf0, dbuf0, ebuf0, sbuf1, dbuf1, ebuf1,
                    wl_src, wl_row, wl_e, gbuf0, gbuf1, csem, gsem):
    sid = lax.axis_index("s")
    wid = lax.axis_index("c") * NS + sid
    lo = wid * NPW
    iota = lax.iota(jnp.int32, L)

    zf = jnp.zeros((L,), jnp.float32)
    zi = jnp.zeros((L,), jnp.int32)
    ninf = jnp.full((L,), -1e30, jnp.float32)

    def init_rows(r, c):
        for cc in range(8):
            maxacc[r, pl.ds(cc * L, L)] = ninf
            sumacc[r, pl.ds(cc * L, L)] = zf
        return c
    lax.fori_loop(0, NPW, init_rows, 0, unroll=False)

    def init_deg(g, c):
        degacc[pl.ds(g * L, L)] = zf
        return c
    lax.fori_loop(0, NPW // L, init_deg, 0, unroll=False)

    def init_wl(g, c):
        wl_src[pl.ds(g * L, L)] = zi
        return c
    lax.fori_loop(0, WLCAP // L, init_wl, 0, unroll=False)

    bufs0 = (sbuf0, dbuf0, ebuf0)
    bufs1 = (sbuf1, dbuf1, ebuf1)

    def issue_chunk(ch, bufs):
        off = ch * CS
        pltpu.async_copy(src_hbm.at[pl.ds(off, CS)], bufs[0], csem)
        pltpu.async_copy(dst_hbm.at[pl.ds(off, CS)], bufs[1], csem)
        pltpu.async_copy(e_hbm.at[pl.ds(off, CS)], bufs[2], csem)

    def wait_chunk(bufs):
        pltpu.make_async_copy(src_hbm.at[pl.ds(0, CS)], bufs[0], csem).wait()
        pltpu.make_async_copy(src_hbm.at[pl.ds(0, CS)], bufs[1], csem).wait()
        pltpu.make_async_copy(e_hbm.at[pl.ds(0, CS)], bufs[2], csem).wait()

    colvs = [jnp.full((L,), cc * L, jnp.int32) + iota for cc in range(8)]

    def edge_work(wl_base, j, gb):
        jv = jnp.full((L,), wl_base + j, jnp.int32)
        rowv = plsc.load_gather(wl_row, [jv])
        evv = plsc.load_gather(wl_e, [jv])
        olds = [plsc.load_gather(maxacc, [rowv, colvs[cc]]) for cc in range(8)]
        gvs = [gb[j, pl.ds(cc * L, L)] for cc in range(8)]
        g2s = [gb[j, pl.ds(D + cc * L, L)] for cc in range(8)]
        for cc in range(8):
            plsc.store_scatter(maxacc, [rowv, colvs[cc]],
                               jnp.maximum(olds[cc], evv * gvs[cc]))
        for cc in range(8):
            plsc.addupdate_scatter(sumacc, [rowv, colvs[cc]], evv * g2s[cc])

    def do_chunk(bufs, carry):
        sb, db, eb = bufs

        def scang(g, wpv):
            dv = db[pl.ds(g * L, L)]
            m = (dv >= lo) & (dv < lo + NPW)
            sv = sb[pl.ds(g * L, L)]
            ev = eb[pl.ds(g * L, L)]
            rv = dv - lo
            cum = plsc.cumsum(jnp.where(m, 1, 0))
            cntv = plsc.all_reduce_population_count(m)
            posv = wpv + cum - 1
            plsc.store_scatter(wl_src, [posv], sv, mask=m)
            plsc.store_scatter(wl_row, [posv], rv, mask=m)
            plsc.store_scatter(wl_e, [posv], ev, mask=m)
            plsc.addupdate_scatter(degacc, [rv],
                                   jnp.ones((L,), jnp.float32), mask=m)
            return wpv + cntv

        halfv = lax.fori_loop(0, CS // (2 * L), scang, carry, unroll=4)
        nsub0 = halfv[0] // G

        @pl.when(nsub0 > 0)
        def _():
            pltpu.async_copy(th_hbm.at[wl_src.at[pl.ds(0, G)]], gbuf0, gsem)

        totalv = lax.fori_loop(CS // (2 * L), CS // L, scang, halfv, unroll=4)
        total = totalv[0]
        nsub = total // G

        def issue_gather(q, gb):
            pltpu.async_copy(th_hbm.at[wl_src.at[pl.ds(q * G, G)]], gb, gsem)

        def wait_gather(gb):
            pltpu.make_async_copy(th_hbm.at[wl_src.at[pl.ds(0, G)]], gb,
                                  gsem).wait()

        def run_edges(q, gb):
            def edge(j, c2):
                edge_work(q * G, j, gb)
                return c2
            lax.fori_loop(0, G, edge, 0, unroll=2)

        @pl.when((nsub > 0) & (nsub0 == 0))
        def _():
            issue_gather(0, gbuf0)

        def subpair(h, c):
            q = h * 2
            wait_gather(gbuf0)

            @pl.when(q + 1 < nsub)
            def _():
                issue_gather(q + 1, gbuf1)
            run_edges(q, gbuf0)

            @pl.when(q + 1 < nsub)
            def _():
                wait_gather(gbuf1)

                @pl.when(q + 2 < nsub)
                def _():
                    issue_gather(q + 2, gbuf0)
                run_edges(q + 1, gbuf1)
            return c
        lax.fori_loop(0, (nsub + 1) // 2, subpair, 0, unroll=False)

        rem = total - nsub * G
        s0 = nsub * G
        for t in range(G // L):
            o = t * L
            a = wl_src[pl.ds(s0 + o, L)]
            b = wl_row[pl.ds(s0 + o, L)]
            cvals = wl_e[pl.ds(s0 + o, L)]
            wl_src[pl.ds(o, L)] = a
            wl_row[pl.ds(o, L)] = b
            wl_e[pl.ds(o, L)] = cvals
        return totalv - nsub * G

    issue_chunk(0, bufs0)

    def chunk_pair(ch, carry_in):
        wait_chunk(bufs0)
        issue_chunk(ch + 1, bufs1)
        c1 = do_chunk(bufs0, carry_in)
        wait_chunk(bufs1)

        @pl.when(ch + 2 < NCH)
        def _():
            issue_chunk(ch + 2, bufs0)
        return do_chunk(bufs1, c1)

    carryv = pl.loop(0, NCH, step=2,
                     init_carry=jnp.zeros((L,), jnp.int32))(chunk_pair)
    carry = carryv[0]

    pltpu.async_copy(th_hbm.at[wl_src.at[pl.ds(0, G)]], gbuf0, gsem).wait()

    def edge_fin(j, c):
        edge_work(0, j, gbuf0)
        return c
    lax.fori_loop(0, carry, edge_fin, 0, unroll=False)

    pltpu.sync_copy(maxacc, maxout.at[pl.ds(lo, NPW)])
    pltpu.sync_copy(sumacc, sumout.at[pl.ds(lo, NPW)])
    pltpu.sync_copy(degacc, degout.at[pl.ds(lo, NPW)])


def _stage_d(src, dst, e, th):
    mesh = plsc.VectorSubcoreMesh(core_axis_name="c", subcore_axis_name="s")
    f = pl.kernel(
        _stage_d_kernel,
        out_type=[
            jax.ShapeDtypeStruct((NPAD, D), jnp.float32),
            jax.ShapeDtypeStruct((NPAD, D), jnp.float32),
            jax.ShapeDtypeStruct((NPAD,), jnp.float32),
        ],
        mesh=mesh,
        compiler_params=pltpu.CompilerParams(needs_layout_passes=False),
        scratch_types=[
            pltpu.VMEM((NPW, D), jnp.float32),
            pltpu.VMEM((NPW, D), jnp.float32),
            pltpu.VMEM((NPW,), jnp.float32),
            pltpu.VMEM((CS,), jnp.int32),
            pltpu.VMEM((CS,), jnp.int32),
            pltpu.VMEM((CS,), jnp.float32),
            pltpu.VMEM((CS,), jnp.int32),
            pltpu.VMEM((CS,), jnp.int32),
            pltpu.VMEM((CS,), jnp.float32),
            pltpu.VMEM((WLCAP,), jnp.int32),
            pltpu.VMEM((WLCAP,), jnp.int32),
            pltpu.VMEM((WLCAP,), jnp.float32),
            pltpu.VMEM((G, 2 * D), jnp.float32),
            pltpu.VMEM((G, 2 * D), jnp.float32),
            pltpu.SemaphoreType.DMA,
            pltpu.SemaphoreType.DMA,
        ],
    )
    return f(src, dst, e, th)


def _stage_e_body(mx_ref, sm_ref, deg_ref, s_ref, wn_ref, bn_ref,
                  wn2_ref, bn2_ref, wm1_ref, bm1_ref, wm2_ref, bm2_ref,
                  out_ref):
    dg = deg_ref[...]
    neigh = jnp.where(dg > 0, mx_ref[...], 0.0)
    neigh2 = sm_ref[...] / jnp.maximum(dg, 1.0)
    rst = (s_ref[...]
           + jnp.dot(neigh, wn_ref[...], preferred_element_type=jnp.float32) + bn_ref[...]
           + jnp.dot(neigh2, wn2_ref[...], preferred_element_type=jnp.float32) + bn2_ref[...])
    rst = rst + jnp.dot(_gelu(rst), wm1_ref[...], preferred_element_type=jnp.float32) + bm1_ref[...]
    rst = rst + jnp.dot(_gelu(rst), wm2_ref[...], preferred_element_type=jnp.float32) + bm2_ref[...]
    out_ref[...] = rst


def _stage_e(mx, sm, deg, s, wn, bn, wn2, bn2, wm1, bm1, wm2, bm2):
    blk = 2000
    grid = (N // blk,)
    wfull = lambda: pl.BlockSpec((D, D), lambda i: (0, 0))
    bfull = lambda: pl.BlockSpec((1, D), lambda i: (0, 0))
    return pl.pallas_call(
        _stage_e_body,
        grid=grid,
        in_specs=[
            pl.BlockSpec((blk, D), lambda i: (i, 0)),
            pl.BlockSpec((blk, D), lambda i: (i, 0)),
            pl.BlockSpec((blk, 1), lambda i: (i, 0)),
            pl.BlockSpec((blk, D), lambda i: (i, 0)),
            wfull(), bfull(), wfull(), bfull(),
            wfull(), bfull(), wfull(), bfull(),
        ],
        out_specs=pl.BlockSpec((blk, D), lambda i: (i, 0)),
        out_shape=jax.ShapeDtypeStruct((N, D), jnp.float32),
    )(mx, sm, deg, s, wn, bn, wn2, bn2, wm1, bm1, wm2, bm2)


_DBG_B = False
_DBG_D = False

def kernel(feat, edge_index, params):
    wsrc, bsrc = _eff_w(params["atten_src"])
    wdst, bdst = _eff_w(params["atten_dst"])
    wsub, bsub = _eff_w(params["atten_sub"])
    wmul, bmul = _eff_w(params["atten_mul"])
    wout, bout = _eff_w(params["atten_out"])
    wp, bp = _eff_w(params["fc_pool"])
    wp2, bp2 = _eff_w(params["fc_pool2"])
    wself, bself = _eff_w(params["fc_self"])
    wn, bn = _eff_w(params["fc_neigh"])
    wn2, bn2 = _eff_w(params["fc_neigh2"])
    btot = (bsrc + bdst + bsub + bmul)[None, :]

    eye = jnp.eye(D, dtype=jnp.float32)
    w1cat = jnp.concatenate([eye, (wsrc + wsub).T], axis=1)
    w2cat = jnp.concatenate([eye, (wdst - wsub).T], axis=1)
    whcat = jnp.concatenate([wp.T, wp2.T], axis=1)
    bhcat = jnp.concatenate([bp, bp2])[None, :]

    tb1, tb2, th, s = _stage_a(feat, w1cat, w2cat, whcat, bhcat,
                               wself.T, bself[None, :])

    src = edge_index[0]
    dst = edge_index[1]
    if _DBG_B:
        ab = jnp.concatenate([tb1[src, :D] * tb2[dst, :D],
                              tb1[src, D:] + tb2[dst, D:]], axis=1)
    else:
        ab = _stage_b(src, dst, tb1, tb2)

    e2 = _stage_c(ab, wmul.T, btot, wout.T, bout[None, None, 0])
    e = e2.reshape(E)

    if _DBG_D:
        mx = jax.ops.segment_max(e[:, None] * th[src, :D], dst, num_segments=NPAD)
        mx = jnp.where(jnp.isfinite(mx), mx, -1e30)
        sm = jax.ops.segment_sum(e[:, None] * th[src, D:], dst, num_segments=NPAD)
        deg = jax.ops.segment_sum(jnp.ones((E,), jnp.float32), dst, num_segments=NPAD)
    else:
        mx, sm, deg = _stage_d(src, dst, e, th)
    deg = deg.reshape(NPAD, 1)

    wmlp1, bmlp1 = _eff_w(params["out_mlp"][0])
    wmlp2, bmlp2 = _eff_w(params["out_mlp"][1])
    rst = _stage_e(mx, sm, deg, s, wn.T, bn[None, :], wn2.T, bn2[None, :],
                   wmlp1.T, bmlp1[None, :], wmlp2.T, bmlp2[None, :])
    return rst


# speedup vs baseline: 2.4409x; 1.0010x over previous
import jax
import jax.numpy as jnp
from jax import lax
from jax.experimental import pallas as pl
from jax.experimental.pallas import tpu as pltpu
from jax.experimental.pallas import tpu_sc as plsc

N = 10000
E = 320000
D = 128

NC = 2
NS = 16
NW = NC * NS
L = 16

NPW = 320
NPAD = NW * NPW
EPW = E // NW
KB = 80
NB_CH = EPW // KB
CS = 2000
G = 32
WLCAP = CS + G


def _gelu(x):
    return x * 0.5 * (1.0 + lax.erf(x * 0.7071067811865476))


def _eff_w(p):
    V, g, b = p["v"], p["g"], p["b"]
    norm = jnp.sqrt(jnp.sum(V * V, axis=1, keepdims=True))
    return (g * V / norm), b


def _stage_a_body(feat_ref, w1_ref, w2_ref, wh_ref, bh_ref, ws_ref, bs_ref,
                  tb1_ref, tb2_ref, th_ref, s_ref):
    f = feat_ref[...]
    tb1_ref[...] = jnp.dot(f, w1_ref[...], preferred_element_type=jnp.float32)
    tb2_ref[...] = jnp.dot(f, w2_ref[...], preferred_element_type=jnp.float32)
    th_ref[...] = _gelu(jnp.dot(f, wh_ref[...], preferred_element_type=jnp.float32)
                        + bh_ref[...])
    s_ref[...] = jnp.dot(f, ws_ref[...], preferred_element_type=jnp.float32) + bs_ref[...]


def _stage_a(feat, w1cat, w2cat, whcat, bhcat, wself, bself):
    blk = 2000
    grid = (N // blk,)
    full = lambda shape: pl.BlockSpec(shape, lambda i: (0, 0))
    return pl.pallas_call(
        _stage_a_body,
        grid=grid,
        in_specs=[
            pl.BlockSpec((blk, D), lambda i: (i, 0)),
            full((D, 2 * D)), full((D, 2 * D)), full((D, 2 * D)),
            full((1, 2 * D)), full((D, D)), full((1, D)),
        ],
        out_specs=[
            pl.BlockSpec((blk, 2 * D), lambda i: (i, 0)),
            pl.BlockSpec((blk, 2 * D), lambda i: (i, 0)),
            pl.BlockSpec((blk, 2 * D), lambda i: (i, 0)),
            pl.BlockSpec((blk, D), lambda i: (i, 0)),
        ],
        out_shape=[
            jax.ShapeDtypeStruct((N, 2 * D), jnp.float32),
            jax.ShapeDtypeStruct((N, 2 * D), jnp.float32),
            jax.ShapeDtypeStruct((N, 2 * D), jnp.float32),
            jax.ShapeDtypeStruct((N, D), jnp.float32),
        ],
    )(feat, w1cat, w2cat, whcat, bhcat, wself, bself)


def _stage_b_kernel(src_hbm, dst_hbm, tb1_hbm, tb2_hbm, out_hbm,
                    sidx, didx, abuf, bbuf, obuf, sema0, semb0, sema1, semb1,
                    osem):
    wid = lax.axis_index("s") * NC + lax.axis_index("c")
    base = wid * EPW
    gsems = ((sema0, semb0), (sema1, semb1))

    pltpu.sync_copy(src_hbm.at[pl.ds(base, EPW)], sidx)
    pltpu.sync_copy(dst_hbm.at[pl.ds(base, EPW)], didx)

    def issue(j, t):
        pltpu.async_copy(tb1_hbm.at[sidx.at[pl.ds(j * KB, KB)]],
                         abuf.at[t], gsems[t][0])
        pltpu.async_copy(tb2_hbm.at[didx.at[pl.ds(j * KB, KB)]],
                         bbuf.at[t], gsems[t][1])

    def wait_slot(j, t):
        pltpu.make_async_copy(tb1_hbm.at[sidx.at[pl.ds(j * KB, KB)]],
                              abuf.at[t], gsems[t][0]).wait()
        pltpu.make_async_copy(tb2_hbm.at[didx.at[pl.ds(j * KB, KB)]],
                              bbuf.at[t], gsems[t][1]).wait()

    def compute_out(j, t):
        @pl.when(j > 0)
        def _():
            pltpu.make_async_copy(obuf, out_hbm.at[pl.ds(base, KB)],
                                  osem).wait()

        def row(k, c2):
            avs = [abuf[t, k, pl.ds(r * L, L)] for r in range(16)]
            bvs = [bbuf[t, k, pl.ds(r * L, L)] for r in range(16)]
            for r in range(16):
                obuf[k, pl.ds(r * L, L)] = (avs[r] * bvs[r] if r < 8
                                            else avs[r] + bvs[r])
            return c2
        lax.fori_loop(0, KB, row, 0, unroll=2)
        pltpu.async_copy(obuf, out_hbm.at[pl.ds(base + j * KB, KB)], osem)

    issue(0, 0)

    @pl.loop(0, NB_CH - 1, step=2)
    def chunk_pair(j):
        wait_slot(j, 0)
        issue(j + 1, 1)
        compute_out(j, 0)
        wait_slot(j + 1, 1)

        @pl.when(j + 2 < NB_CH)
        def _():
            issue(j + 2, 0)
        compute_out(j + 1, 1)

    wait_slot(NB_CH - 1, 0)
    compute_out(NB_CH - 1, 0)
    pltpu.make_async_copy(obuf, out_hbm.at[pl.ds(base, KB)], osem).wait()


def _stage_b(src, dst, tb1, tb2):
    mesh = plsc.VectorSubcoreMesh(core_axis_name="c", subcore_axis_name="s")
    f = pl.kernel(
        _stage_b_kernel,
        out_type=jax.ShapeDtypeStruct((E, 2 * D), jnp.float32),
        mesh=mesh,
        compiler_params=pltpu.CompilerParams(needs_layout_passes=False),
        scratch_types=[
            pltpu.VMEM((EPW,), jnp.int32),
            pltpu.VMEM((EPW,), jnp.int32),
            pltpu.VMEM((2, KB, 2 * D), jnp.float32),
            pltpu.VMEM((2, KB, 2 * D), jnp.float32),
            pltpu.VMEM((KB, 2 * D), jnp.float32),
            pltpu.SemaphoreType.DMA,
            pltpu.SemaphoreType.DMA,
            pltpu.SemaphoreType.DMA,
            pltpu.SemaphoreType.DMA,
            pltpu.SemaphoreType.DMA,
        ],
    )
    return f(src, dst, tb1, tb2)


def _stage_c_body(ab_ref, wm_ref, btot_ref, wo_ref, bo_ref, out_ref):
    ab = ab_ref[...]
    pre = (jnp.dot(ab[:, :D], wm_ref[...], preferred_element_type=jnp.float32)
           + ab[:, D:] + btot_ref[...])
    x2 = _gelu(pre)
    y = jnp.dot(x2, wo_ref[...], preferred_element_type=jnp.float32) + bo_ref[...]
    out_ref[...] = jnp.where(y > 0, y, 0.2 * y)


def _stage_c(ab, wm, btot, wout2, bout2):
    blk = 3200
    grid = (E // blk,)
    return pl.pallas_call(
        _stage_c_body,
        grid=grid,
        in_specs=[
            pl.BlockSpec((blk, 2 * D), lambda i: (i, 0)),
            pl.BlockSpec((D, D), lambda i: (0, 0)),
            pl.BlockSpec((1, D), lambda i: (0, 0)),
            pl.BlockSpec((D, 1), lambda i: (0, 0)),
            pl.BlockSpec((1, 1), lambda i: (0, 0)),
        ],
        out_specs=pl.BlockSpec((blk, 1), lambda i: (i, 0)),
        out_shape=jax.ShapeDtypeStruct((E, 1), jnp.float32),
    )(ab, wm, btot, wout2, bout2)


NCH = E // CS


def _stage_d_kernel(src_hbm, dst_hbm, e_hbm, th_hbm,
                    maxout, sumout, degout,
                    maxacc, sumacc, degacc,
                    sbuf0, dbuf0, ebuf0, sbuf1, dbuf1, ebuf1,
                    wl_src, wl_row, wl_e, gbuf0, gbuf1, csem, gsem):
    sid = lax.axis_index("s")
    wid = lax.axis_index("c") * NS + sid
    lo = wid * NPW
    iota = lax.iota(jnp.int32, L)

    zf = jnp.zeros((L,), jnp.float32)
    zi = jnp.zeros((L,), jnp.int32)
    ninf = jnp.full((L,), -1e30, jnp.float32)

    def init_rows(r, c):
        for cc in range(8):
            maxacc[r, pl.ds(cc * L, L)] = ninf
            sumacc[r, pl.ds(cc * L, L)] = zf
        return c
    lax.fori_loop(0, NPW, init_rows, 0, unroll=False)

    def init_deg(g, c):
        degacc[pl.ds(g * L, L)] = zf
        return c
    lax.fori_loop(0, NPW // L, init_deg, 0, unroll=False)

    def init_wl(g, c):
        wl_src[pl.ds(g * L, L)] = zi
        return c
    lax.fori_loop(0, WLCAP // L, init_wl, 0, unroll=False)

    bufs0 = (sbuf0, dbuf0, ebuf0)
    bufs1 = (sbuf1, dbuf1, ebuf1)

    def issue_chunk(ch, bufs):
        off = ch * CS
        pltpu.async_copy(src_hbm.at[pl.ds(off, CS)], bufs[0], csem)
        pltpu.async_copy(dst_hbm.at[pl.ds(off, CS)], bufs[1], csem)
        pltpu.async_copy(e_hbm.at[pl.ds(off, CS)], bufs[2], csem)

    def wait_chunk(bufs):
        pltpu.make_async_copy(src_hbm.at[pl.ds(0, CS)], bufs[0], csem).wait()
        pltpu.make_async_copy(src_hbm.at[pl.ds(0, CS)], bufs[1], csem).wait()
        pltpu.make_async_copy(e_hbm.at[pl.ds(0, CS)], bufs[2], csem).wait()

    colvs = [jnp.full((L,), cc * L, jnp.int32) + iota for cc in range(8)]

    def edge_work(wl_base, j, gb):
        jv = jnp.full((L,), wl_base + j, jnp.int32)
        rowv = plsc.load_gather(wl_row, [jv])
        evv = plsc.load_gather(wl_e, [jv])
        olds = [plsc.load_gather(maxacc, [rowv, colvs[cc]]) for cc in range(8)]
        gvs = [gb[j, pl.ds(cc * L, L)] for cc in range(8)]
        g2s = [gb[j, pl.ds(D + cc * L, L)] for cc in range(8)]
        for cc in range(8):
            plsc.store_scatter(maxacc, [rowv, colvs[cc]],
                               jnp.maximum(olds[cc], evv * gvs[cc]))
        for cc in range(8):
            plsc.addupdate_scatter(sumacc, [rowv, colvs[cc]], evv * g2s[cc])

    def do_chunk(bufs, carry):
        sb, db, eb = bufs

        def scang(g, wpv):
            dv = db[pl.ds(g * L, L)]
            m = (dv >= lo) & (dv < lo + NPW)
            sv = sb[pl.ds(g * L, L)]
            ev = eb[pl.ds(g * L, L)]
            rv = dv - lo
            cum = plsc.cumsum(jnp.where(m, 1, 0))
            cntv = plsc.all_reduce_population_count(m)
            posv = wpv + cum - 1
            plsc.store_scatter(wl_src, [posv], sv, mask=m)
            plsc.store_scatter(wl_row, [posv], rv, mask=m)
            plsc.store_scatter(wl_e, [posv], ev, mask=m)
            plsc.addupdate_scatter(degacc, [rv],
                                   jnp.ones((L,), jnp.float32), mask=m)
            return wpv + cntv

        halfv = lax.fori_loop(0, CS // (2 * L), scang, carry, unroll=4)
        nsub0 = halfv[0] // G

        @pl.when(nsub0 > 0)
        def _():
            pltpu.async_copy(th_hbm.at[wl_src.at[pl.ds(0, G)]], gbuf0, gsem)

        totalv = lax.fori_loop(CS // (2 * L), CS // L, scang, halfv, unroll=4)
        total = totalv[0]
        nsub = total // G

        def issue_gather(q, gb):
            pltpu.async_copy(th_hbm.at[wl_src.at[pl.ds(q * G, G)]], gb, gsem)

        def wait_gather(gb):
            pltpu.make_async_copy(th_hbm.at[wl_src.at[pl.ds(0, G)]], gb,
                                  gsem).wait()

        def run_edges(q, gb):
            def edge(j, c2):
                edge_work(q * G, j, gb)
                return c2
            lax.fori_loop(0, G, edge, 0, unroll=2)

        @pl.when((nsub > 0) & (nsub0 == 0))
        def _():
            issue_gather(0, gbuf0)

        def subpair(h, c):
            q = h * 2
            wait_gather(gbuf0)

            @pl.when(q + 1 < nsub)
            def _():
                issue_gather(q + 1, gbuf1)
            run_edges(q, gbuf0)

            @pl.when(q + 1 < nsub)
            def _():
                wait_gather(gbuf1)

                @pl.when(q + 2 < nsub)
                def _():
                    issue_gather(q + 2, gbuf0)
                run_edges(q + 1, gbuf1)
            return c
        lax.fori_loop(0, (nsub + 1) // 2, subpair, 0, unroll=False)

        rem = total - nsub * G
        s0 = nsub * G
        for t in range(G // L):
            o = t * L
            a = wl_src[pl.ds(s0 + o, L)]
            b = wl_row[pl.ds(s0 + o, L)]
            cvals = wl_e[pl.ds(s0 + o, L)]
            wl_src[pl.ds(o, L)] = a
            wl_row[pl.ds(o, L)] = b
            wl_e[pl.ds(o, L)] = cvals
        return totalv - nsub * G

    issue_chunk(0, bufs0)

    def chunk_pair(ch, carry_in):
        wait_chunk(bufs0)
        issue_chunk(ch + 1, bufs1)
        c1 = do_chunk(bufs0, carry_in)
        wait_chunk(bufs1)

        @pl.when(ch + 2 < NCH)
        def _():
            issue_chunk(ch + 2, bufs0)
        return do_chunk(bufs1, c1)

    carryv = pl.loop(0, NCH, step=2,
                     init_carry=jnp.zeros((L,), jnp.int32))(chunk_pair)
    carry = carryv[0]

    pltpu.async_copy(th_hbm.at[wl_src.at[pl.ds(0, G)]], gbuf0, gsem).wait()

    def edge_fin(j, c):
        edge_work(0, j, gbuf0)
        return c
    lax.fori_loop(0, carry, edge_fin, 0, unroll=False)

    pltpu.sync_copy(maxacc, maxout.at[pl.ds(lo, NPW)])
    pltpu.sync_copy(sumacc, sumout.at[pl.ds(lo, NPW)])
    pltpu.sync_copy(degacc, degout.at[pl.ds(lo, NPW)])


def _stage_d(src, dst, e, th):
    mesh = plsc.VectorSubcoreMesh(core_axis_name="c", subcore_axis_name="s")
    f = pl.kernel(
        _stage_d_kernel,
        out_type=[
            jax.ShapeDtypeStruct((NPAD, D), jnp.float32),
            jax.ShapeDtypeStruct((NPAD, D), jnp.float32),
            jax.ShapeDtypeStruct((NPAD,), jnp.float32),
        ],
        mesh=mesh,
        compiler_params=pltpu.CompilerParams(needs_layout_passes=False),
        scratch_types=[
            pltpu.VMEM((NPW, D), jnp.float32),
            pltpu.VMEM((NPW, D), jnp.float32),
            pltpu.VMEM((NPW,), jnp.float32),
            pltpu.VMEM((CS,), jnp.int32),
            pltpu.VMEM((CS,), jnp.int32),
            pltpu.VMEM((CS,), jnp.float32),
            pltpu.VMEM((CS,), jnp.int32),
            pltpu.VMEM((CS,), jnp.int32),
            pltpu.VMEM((CS,), jnp.float32),
            pltpu.VMEM((WLCAP,), jnp.int32),
            pltpu.VMEM((WLCAP,), jnp.int32),
            pltpu.VMEM((WLCAP,), jnp.float32),
            pltpu.VMEM((G, 2 * D), jnp.float32),
            pltpu.VMEM((G, 2 * D), jnp.float32),
            pltpu.SemaphoreType.DMA,
            pltpu.SemaphoreType.DMA,
        ],
    )
    return f(src, dst, e, th)


def _stage_e_body(mx_ref, sm_ref, deg_ref, s_ref, wn_ref, bn_ref,
                  wn2_ref, bn2_ref, wm1_ref, bm1_ref, wm2_ref, bm2_ref,
                  out_ref):
    dg = deg_ref[...]
    neigh = jnp.where(dg > 0, mx_ref[...], 0.0)
    neigh2 = sm_ref[...] / jnp.maximum(dg, 1.0)
    rst = (s_ref[...]
           + jnp.dot(neigh, wn_ref[...], preferred_element_type=jnp.float32) + bn_ref[...]
           + jnp.dot(neigh2, wn2_ref[...], preferred_element_type=jnp.float32) + bn2_ref[...])
    rst = rst + jnp.dot(_gelu(rst), wm1_ref[...], preferred_element_type=jnp.float32) + bm1_ref[...]
    rst = rst + jnp.dot(_gelu(rst), wm2_ref[...], preferred_element_type=jnp.float32) + bm2_ref[...]
    out_ref[...] = rst


def _stage_e(mx, sm, deg, s, wn, bn, wn2, bn2, wm1, bm1, wm2, bm2):
    blk = 2000
    grid = (N // blk,)
    wfull = lambda: pl.BlockSpec((D, D), lambda i: (0, 0))
    bfull = lambda: pl.BlockSpec((1, D), lambda i: (0, 0))
    return pl.pallas_call(
        _stage_e_body,
        grid=grid,
        in_specs=[
            pl.BlockSpec((blk, D), lambda i: (i, 0)),
            pl.BlockSpec((blk, D), lambda i: (i, 0)),
            pl.BlockSpec((blk, 1), lambda i: (i, 0)),
            pl.BlockSpec((blk, D), lambda i: (i, 0)),
            wfull(), bfull(), wfull(), bfull(),
            wfull(), bfull(), wfull(), bfull(),
        ],
        out_specs=pl.BlockSpec((blk, D), lambda i: (i, 0)),
        out_shape=jax.ShapeDtypeStruct((N, D), jnp.float32),
    )(mx, sm, deg, s, wn, bn, wn2, bn2, wm1, bm1, wm2, bm2)


_DBG_B = False
_DBG_D = False

def kernel(feat, edge_index, params):
    wsrc, bsrc = _eff_w(params["atten_src"])
    wdst, bdst = _eff_w(params["atten_dst"])
    wsub, bsub = _eff_w(params["atten_sub"])
    wmul, bmul = _eff_w(params["atten_mul"])
    wout, bout = _eff_w(params["atten_out"])
    wp, bp = _eff_w(params["fc_pool"])
    wp2, bp2 = _eff_w(params["fc_pool2"])
    wself, bself = _eff_w(params["fc_self"])
    wn, bn = _eff_w(params["fc_neigh"])
    wn2, bn2 = _eff_w(params["fc_neigh2"])
    btot = (bsrc + bdst + bsub + bmul)[None, :]

    eye = jnp.eye(D, dtype=jnp.float32)
    w1cat = jnp.concatenate([eye, (wsrc + wsub).T], axis=1)
    w2cat = jnp.concatenate([eye, (wdst - wsub).T], axis=1)
    whcat = jnp.concatenate([wp.T, wp2.T], axis=1)
    bhcat = jnp.concatenate([bp, bp2])[None, :]

    tb1, tb2, th, s = _stage_a(feat, w1cat, w2cat, whcat, bhcat,
                               wself.T, bself[None, :])

    src = edge_index[0]
    dst = edge_index[1]
    if _DBG_B:
        ab = jnp.concatenate([tb1[src, :D] * tb2[dst, :D],
                              tb1[src, D:] + tb2[dst, D:]], axis=1)
    else:
        ab = _stage_b(src, dst, tb1, tb2)

    e2 = _stage_c(ab, wmul.T, btot, wout.T, bout[None, None, 0])
    e = e2.reshape(E)

    if _DBG_D:
        mx = jax.ops.segment_max(e[:, None] * th[src, :D], dst, num_segments=NPAD)
        mx = jnp.where(jnp.isfinite(mx), mx, -1e30)
        sm = jax.ops.segment_sum(e[:, None] * th[src, D:], dst, num_segments=NPAD)
        deg = jax.ops.segment_sum(jnp.ones((E,), jnp.float32), dst, num_segments=NPAD)
    else:
        mx, sm, deg = _stage_d(src, dst, e, th)
    deg = deg.reshape(NPAD, 1)

    wmlp1, bmlp1 = _eff_w(params["out_mlp"][0])
    wmlp2, bmlp2 = _eff_w(params["out_mlp"][1])
    rst = _stage_e(mx, sm, deg, s, wn.T, bn[None, :], wn2.T, bn2[None, :],
                   wmlp1.T, bmlp1[None, :], wmlp2.T, bmlp2[None, :])
    return rst


# speedup vs baseline: 2.4612x; 1.0083x over previous
import jax
import jax.numpy as jnp
from jax import lax
from jax.experimental import pallas as pl
from jax.experimental.pallas import tpu as pltpu
from jax.experimental.pallas import tpu_sc as plsc

N = 10000
E = 320000
D = 128

NC = 2
NS = 16
NW = NC * NS
L = 16

NPW = 320
NPAD = NW * NPW
EPW = E // NW
KB = 80
NB_CH = EPW // KB
CS = 3200
G = 32
WLCAP = CS + G


def _gelu(x):
    return x * 0.5 * (1.0 + lax.erf(x * 0.7071067811865476))


def _eff_w(p):
    V, g, b = p["v"], p["g"], p["b"]
    norm = jnp.sqrt(jnp.sum(V * V, axis=1, keepdims=True))
    return (g * V / norm), b


def _stage_a_body(feat_ref, w1_ref, w2_ref, wh_ref, bh_ref, ws_ref, bs_ref,
                  tb1_ref, tb2_ref, th_ref, s_ref):
    f = feat_ref[...]
    tb1_ref[...] = jnp.dot(f, w1_ref[...], preferred_element_type=jnp.float32)
    tb2_ref[...] = jnp.dot(f, w2_ref[...], preferred_element_type=jnp.float32)
    th_ref[...] = _gelu(jnp.dot(f, wh_ref[...], preferred_element_type=jnp.float32)
                        + bh_ref[...])
    s_ref[...] = jnp.dot(f, ws_ref[...], preferred_element_type=jnp.float32) + bs_ref[...]


def _stage_a(feat, w1cat, w2cat, whcat, bhcat, wself, bself):
    blk = 2000
    grid = (N // blk,)
    full = lambda shape: pl.BlockSpec(shape, lambda i: (0, 0))
    return pl.pallas_call(
        _stage_a_body,
        grid=grid,
        in_specs=[
            pl.BlockSpec((blk, D), lambda i: (i, 0)),
            full((D, 2 * D)), full((D, 2 * D)), full((D, 2 * D)),
            full((1, 2 * D)), full((D, D)), full((1, D)),
        ],
        out_specs=[
            pl.BlockSpec((blk, 2 * D), lambda i: (i, 0)),
            pl.BlockSpec((blk, 2 * D), lambda i: (i, 0)),
            pl.BlockSpec((blk, 2 * D), lambda i: (i, 0)),
            pl.BlockSpec((blk, D), lambda i: (i, 0)),
        ],
        out_shape=[
            jax.ShapeDtypeStruct((N, 2 * D), jnp.float32),
            jax.ShapeDtypeStruct((N, 2 * D), jnp.float32),
            jax.ShapeDtypeStruct((N, 2 * D), jnp.float32),
            jax.ShapeDtypeStruct((N, D), jnp.float32),
        ],
    )(feat, w1cat, w2cat, whcat, bhcat, wself, bself)


def _stage_b_kernel(src_hbm, dst_hbm, tb1_hbm, tb2_hbm, out_hbm,
                    sidx, didx, abuf, bbuf, obuf, sema0, semb0, sema1, semb1,
                    osem):
    wid = lax.axis_index("s") * NC + lax.axis_index("c")
    base = wid * EPW
    gsems = ((sema0, semb0), (sema1, semb1))

    pltpu.sync_copy(src_hbm.at[pl.ds(base, EPW)], sidx)
    pltpu.sync_copy(dst_hbm.at[pl.ds(base, EPW)], didx)

    def issue(j, t):
        pltpu.async_copy(tb1_hbm.at[sidx.at[pl.ds(j * KB, KB)]],
                         abuf.at[t], gsems[t][0])
        pltpu.async_copy(tb2_hbm.at[didx.at[pl.ds(j * KB, KB)]],
                         bbuf.at[t], gsems[t][1])

    def wait_slot(j, t):
        pltpu.make_async_copy(tb1_hbm.at[sidx.at[pl.ds(j * KB, KB)]],
                              abuf.at[t], gsems[t][0]).wait()
        pltpu.make_async_copy(tb2_hbm.at[didx.at[pl.ds(j * KB, KB)]],
                              bbuf.at[t], gsems[t][1]).wait()

    def compute_out(j, t):
        @pl.when(j > 0)
        def _():
            pltpu.make_async_copy(obuf, out_hbm.at[pl.ds(base, KB)],
                                  osem).wait()

        def row(k, c2):
            avs = [abuf[t, k, pl.ds(r * L, L)] for r in range(16)]
            bvs = [bbuf[t, k, pl.ds(r * L, L)] for r in range(16)]
            for r in range(16):
                obuf[k, pl.ds(r * L, L)] = (avs[r] * bvs[r] if r < 8
                                            else avs[r] + bvs[r])
            return c2
        lax.fori_loop(0, KB, row, 0, unroll=2)
        pltpu.async_copy(obuf, out_hbm.at[pl.ds(base + j * KB, KB)], osem)

    issue(0, 0)

    @pl.loop(0, NB_CH - 1, step=2)
    def chunk_pair(j):
        wait_slot(j, 0)
        issue(j + 1, 1)
        compute_out(j, 0)
        wait_slot(j + 1, 1)

        @pl.when(j + 2 < NB_CH)
        def _():
            issue(j + 2, 0)
        compute_out(j + 1, 1)

    wait_slot(NB_CH - 1, 0)
    compute_out(NB_CH - 1, 0)
    pltpu.make_async_copy(obuf, out_hbm.at[pl.ds(base, KB)], osem).wait()


def _stage_b(src, dst, tb1, tb2):
    mesh = plsc.VectorSubcoreMesh(core_axis_name="c", subcore_axis_name="s")
    f = pl.kernel(
        _stage_b_kernel,
        out_type=jax.ShapeDtypeStruct((E, 2 * D), jnp.float32),
        mesh=mesh,
        compiler_params=pltpu.CompilerParams(needs_layout_passes=False),
        scratch_types=[
            pltpu.VMEM((EPW,), jnp.int32),
            pltpu.VMEM((EPW,), jnp.int32),
            pltpu.VMEM((2, KB, 2 * D), jnp.float32),
            pltpu.VMEM((2, KB, 2 * D), jnp.float32),
            pltpu.VMEM((KB, 2 * D), jnp.float32),
            pltpu.SemaphoreType.DMA,
            pltpu.SemaphoreType.DMA,
            pltpu.SemaphoreType.DMA,
            pltpu.SemaphoreType.DMA,
            pltpu.SemaphoreType.DMA,
        ],
    )
    return f(src, dst, tb1, tb2)


def _stage_c_body(ab_ref, wm_ref, btot_ref, wo_ref, bo_ref, out_ref):
    ab = ab_ref[...]
    pre = (jnp.dot(ab[:, :D], wm_ref[...], preferred_element_type=jnp.float32)
           + ab[:, D:] + btot_ref[...])
    x2 = _gelu(pre)
    y = jnp.dot(x2, wo_ref[...], preferred_element_type=jnp.float32) + bo_ref[...]
    out_ref[...] = jnp.where(y > 0, y, 0.2 * y)


def _stage_c(ab, wm, btot, wout2, bout2):
    blk = 3200
    grid = (E // blk,)
    return pl.pallas_call(
        _stage_c_body,
        grid=grid,
        in_specs=[
            pl.BlockSpec((blk, 2 * D), lambda i: (i, 0)),
            pl.BlockSpec((D, D), lambda i: (0, 0)),
            pl.BlockSpec((1, D), lambda i: (0, 0)),
            pl.BlockSpec((D, 1), lambda i: (0, 0)),
            pl.BlockSpec((1, 1), lambda i: (0, 0)),
        ],
        out_specs=pl.BlockSpec((blk, 1), lambda i: (i, 0)),
        out_shape=jax.ShapeDtypeStruct((E, 1), jnp.float32),
    )(ab, wm, btot, wout2, bout2)


NCH = E // CS


def _stage_d_kernel(src_hbm, dst_hbm, e_hbm, th_hbm,
                    maxout, sumout, degout,
                    maxacc, sumacc, degacc,
                    sbuf0, dbuf0, ebuf0, sbuf1, dbuf1, ebuf1,
                    wl_src, wl_row, wl_e, gbuf0, gbuf1, csem, gsem):
    sid = lax.axis_index("s")
    wid = lax.axis_index("c") * NS + sid
    lo = wid * NPW
    iota = lax.iota(jnp.int32, L)

    zf = jnp.zeros((L,), jnp.float32)
    zi = jnp.zeros((L,), jnp.int32)
    ninf = jnp.full((L,), -1e30, jnp.float32)

    def init_rows(r, c):
        for cc in range(8):
            maxacc[r, pl.ds(cc * L, L)] = ninf
            sumacc[r, pl.ds(cc * L, L)] = zf
        return c
    lax.fori_loop(0, NPW, init_rows, 0, unroll=False)

    def init_deg(g, c):
        degacc[pl.ds(g * L, L)] = zf
        return c
    lax.fori_loop(0, NPW // L, init_deg, 0, unroll=False)

    def init_wl(g, c):
        wl_src[pl.ds(g * L, L)] = zi
        return c
    lax.fori_loop(0, WLCAP // L, init_wl, 0, unroll=False)

    bufs0 = (sbuf0, dbuf0, ebuf0)
    bufs1 = (sbuf1, dbuf1, ebuf1)

    def issue_chunk(ch, bufs):
        off = ch * CS
        pltpu.async_copy(src_hbm.at[pl.ds(off, CS)], bufs[0], csem)
        pltpu.async_copy(dst_hbm.at[pl.ds(off, CS)], bufs[1], csem)
        pltpu.async_copy(e_hbm.at[pl.ds(off, CS)], bufs[2], csem)

    def wait_chunk(bufs):
        pltpu.make_async_copy(src_hbm.at[pl.ds(0, CS)], bufs[0], csem).wait()
        pltpu.make_async_copy(src_hbm.at[pl.ds(0, CS)], bufs[1], csem).wait()
        pltpu.make_async_copy(e_hbm.at[pl.ds(0, CS)], bufs[2], csem).wait()

    colvs = [jnp.full((L,), cc * L, jnp.int32) + iota for cc in range(8)]

    def edge_work(wl_base, j, gb):
        jv = jnp.full((L,), wl_base + j, jnp.int32)
        rowv = plsc.load_gather(wl_row, [jv])
        evv = plsc.load_gather(wl_e, [jv])
        olds = [plsc.load_gather(maxacc, [rowv, colvs[cc]]) for cc in range(8)]
        gvs = [gb[j, pl.ds(cc * L, L)] for cc in range(8)]
        g2s = [gb[j, pl.ds(D + cc * L, L)] for cc in range(8)]
        for cc in range(8):
            plsc.store_scatter(maxacc, [rowv, colvs[cc]],
                               jnp.maximum(olds[cc], evv * gvs[cc]))
        for cc in range(8):
            plsc.addupdate_scatter(sumacc, [rowv, colvs[cc]], evv * g2s[cc])

    def do_chunk(bufs, carry):
        sb, db, eb = bufs

        def scang(g, wpv):
            dv = db[pl.ds(g * L, L)]
            m = (dv >= lo) & (dv < lo + NPW)
            sv = sb[pl.ds(g * L, L)]
            ev = eb[pl.ds(g * L, L)]
            rv = dv - lo
            cum = plsc.cumsum(jnp.where(m, 1, 0))
            cntv = plsc.all_reduce_population_count(m)
            posv = wpv + cum - 1
            plsc.store_scatter(wl_src, [posv], sv, mask=m)
            plsc.store_scatter(wl_row, [posv], rv, mask=m)
            plsc.store_scatter(wl_e, [posv], ev, mask=m)
            plsc.addupdate_scatter(degacc, [rv],
                                   jnp.ones((L,), jnp.float32), mask=m)
            return wpv + cntv

        halfv = lax.fori_loop(0, CS // (2 * L), scang, carry, unroll=4)
        nsub0 = halfv[0] // G

        @pl.when(nsub0 > 0)
        def _():
            pltpu.async_copy(th_hbm.at[wl_src.at[pl.ds(0, G)]], gbuf0, gsem)

        totalv = lax.fori_loop(CS // (2 * L), CS // L, scang, halfv, unroll=4)
        total = totalv[0]
        nsub = total // G

        def issue_gather(q, gb):
            pltpu.async_copy(th_hbm.at[wl_src.at[pl.ds(q * G, G)]], gb, gsem)

        def wait_gather(gb):
            pltpu.make_async_copy(th_hbm.at[wl_src.at[pl.ds(0, G)]], gb,
                                  gsem).wait()

        def run_edges(q, gb):
            def edge(j, c2):
                edge_work(q * G, j, gb)
                return c2
            lax.fori_loop(0, G, edge, 0, unroll=2)

        @pl.when((nsub > 0) & (nsub0 == 0))
        def _():
            issue_gather(0, gbuf0)

        def subpair(h, c):
            q = h * 2
            wait_gather(gbuf0)

            @pl.when(q + 1 < nsub)
            def _():
                issue_gather(q + 1, gbuf1)
            run_edges(q, gbuf0)

            @pl.when(q + 1 < nsub)
            def _():
                wait_gather(gbuf1)

                @pl.when(q + 2 < nsub)
                def _():
                    issue_gather(q + 2, gbuf0)
                run_edges(q + 1, gbuf1)
            return c
        lax.fori_loop(0, (nsub + 1) // 2, subpair, 0, unroll=False)

        rem = total - nsub * G
        s0 = nsub * G
        for t in range(G // L):
            o = t * L
            a = wl_src[pl.ds(s0 + o, L)]
            b = wl_row[pl.ds(s0 + o, L)]
            cvals = wl_e[pl.ds(s0 + o, L)]
            wl_src[pl.ds(o, L)] = a
            wl_row[pl.ds(o, L)] = b
            wl_e[pl.ds(o, L)] = cvals
        return totalv - nsub * G

    issue_chunk(0, bufs0)

    def chunk_pair(ch, carry_in):
        wait_chunk(bufs0)
        issue_chunk(ch + 1, bufs1)
        c1 = do_chunk(bufs0, carry_in)
        wait_chunk(bufs1)

        @pl.when(ch + 2 < NCH)
        def _():
            issue_chunk(ch + 2, bufs0)
        return do_chunk(bufs1, c1)

    carryv = pl.loop(0, NCH, step=2,
                     init_carry=jnp.zeros((L,), jnp.int32))(chunk_pair)
    carry = carryv[0]

    pltpu.async_copy(th_hbm.at[wl_src.at[pl.ds(0, G)]], gbuf0, gsem).wait()

    def edge_fin(j, c):
        edge_work(0, j, gbuf0)
        return c
    lax.fori_loop(0, carry, edge_fin, 0, unroll=False)

    pltpu.sync_copy(maxacc, maxout.at[pl.ds(lo, NPW)])
    pltpu.sync_copy(sumacc, sumout.at[pl.ds(lo, NPW)])
    pltpu.sync_copy(degacc, degout.at[pl.ds(lo, NPW)])


def _stage_d(src, dst, e, th):
    mesh = plsc.VectorSubcoreMesh(core_axis_name="c", subcore_axis_name="s")
    f = pl.kernel(
        _stage_d_kernel,
        out_type=[
            jax.ShapeDtypeStruct((NPAD, D), jnp.float32),
            jax.ShapeDtypeStruct((NPAD, D), jnp.float32),
            jax.ShapeDtypeStruct((NPAD,), jnp.float32),
        ],
        mesh=mesh,
        compiler_params=pltpu.CompilerParams(needs_layout_passes=False),
        scratch_types=[
            pltpu.VMEM((NPW, D), jnp.float32),
            pltpu.VMEM((NPW, D), jnp.float32),
            pltpu.VMEM((NPW,), jnp.float32),
            pltpu.VMEM((CS,), jnp.int32),
            pltpu.VMEM((CS,), jnp.int32),
            pltpu.VMEM((CS,), jnp.float32),
            pltpu.VMEM((CS,), jnp.int32),
            pltpu.VMEM((CS,), jnp.int32),
            pltpu.VMEM((CS,), jnp.float32),
            pltpu.VMEM((WLCAP,), jnp.int32),
            pltpu.VMEM((WLCAP,), jnp.int32),
            pltpu.VMEM((WLCAP,), jnp.float32),
            pltpu.VMEM((G, 2 * D), jnp.float32),
            pltpu.VMEM((G, 2 * D), jnp.float32),
            pltpu.SemaphoreType.DMA,
            pltpu.SemaphoreType.DMA,
        ],
    )
    return f(src, dst, e, th)


def _stage_e_body(mx_ref, sm_ref, deg_ref, s_ref, wn_ref, bn_ref,
                  wn2_ref, bn2_ref, wm1_ref, bm1_ref, wm2_ref, bm2_ref,
                  out_ref):
    dg = deg_ref[...]
    neigh = jnp.where(dg > 0, mx_ref[...], 0.0)
    neigh2 = sm_ref[...] / jnp.maximum(dg, 1.0)
    rst = (s_ref[...]
           + jnp.dot(neigh, wn_ref[...], preferred_element_type=jnp.float32) + bn_ref[...]
           + jnp.dot(neigh2, wn2_ref[...], preferred_element_type=jnp.float32) + bn2_ref[...])
    rst = rst + jnp.dot(_gelu(rst), wm1_ref[...], preferred_element_type=jnp.float32) + bm1_ref[...]
    rst = rst + jnp.dot(_gelu(rst), wm2_ref[...], preferred_element_type=jnp.float32) + bm2_ref[...]
    out_ref[...] = rst


def _stage_e(mx, sm, deg, s, wn, bn, wn2, bn2, wm1, bm1, wm2, bm2):
    blk = 2000
    grid = (N // blk,)
    wfull = lambda: pl.BlockSpec((D, D), lambda i: (0, 0))
    bfull = lambda: pl.BlockSpec((1, D), lambda i: (0, 0))
    return pl.pallas_call(
        _stage_e_body,
        grid=grid,
        in_specs=[
            pl.BlockSpec((blk, D), lambda i: (i, 0)),
            pl.BlockSpec((blk, D), lambda i: (i, 0)),
            pl.BlockSpec((blk, 1), lambda i: (i, 0)),
            pl.BlockSpec((blk, D), lambda i: (i, 0)),
            wfull(), bfull(), wfull(), bfull(),
            wfull(), bfull(), wfull(), bfull(),
        ],
        out_specs=pl.BlockSpec((blk, D), lambda i: (i, 0)),
        out_shape=jax.ShapeDtypeStruct((N, D), jnp.float32),
    )(mx, sm, deg, s, wn, bn, wn2, bn2, wm1, bm1, wm2, bm2)


_DBG_B = False
_DBG_D = False

def kernel(feat, edge_index, params):
    wsrc, bsrc = _eff_w(params["atten_src"])
    wdst, bdst = _eff_w(params["atten_dst"])
    wsub, bsub = _eff_w(params["atten_sub"])
    wmul, bmul = _eff_w(params["atten_mul"])
    wout, bout = _eff_w(params["atten_out"])
    wp, bp = _eff_w(params["fc_pool"])
    wp2, bp2 = _eff_w(params["fc_pool2"])
    wself, bself = _eff_w(params["fc_self"])
    wn, bn = _eff_w(params["fc_neigh"])
    wn2, bn2 = _eff_w(params["fc_neigh2"])
    btot = (bsrc + bdst + bsub + bmul)[None, :]

    eye = jnp.eye(D, dtype=jnp.float32)
    w1cat = jnp.concatenate([eye, (wsrc + wsub).T], axis=1)
    w2cat = jnp.concatenate([eye, (wdst - wsub).T], axis=1)
    whcat = jnp.concatenate([wp.T, wp2.T], axis=1)
    bhcat = jnp.concatenate([bp, bp2])[None, :]

    tb1, tb2, th, s = _stage_a(feat, w1cat, w2cat, whcat, bhcat,
                               wself.T, bself[None, :])

    src = edge_index[0]
    dst = edge_index[1]
    if _DBG_B:
        ab = jnp.concatenate([tb1[src, :D] * tb2[dst, :D],
                              tb1[src, D:] + tb2[dst, D:]], axis=1)
    else:
        ab = _stage_b(src, dst, tb1, tb2)

    e2 = _stage_c(ab, wmul.T, btot, wout.T, bout[None, None, 0])
    e = e2.reshape(E)

    if _DBG_D:
        mx = jax.ops.segment_max(e[:, None] * th[src, :D], dst, num_segments=NPAD)
        mx = jnp.where(jnp.isfinite(mx), mx, -1e30)
        sm = jax.ops.segment_sum(e[:, None] * th[src, D:], dst, num_segments=NPAD)
        deg = jax.ops.segment_sum(jnp.ones((E,), jnp.float32), dst, num_segments=NPAD)
    else:
        mx, sm, deg = _stage_d(src, dst, e, th)
    deg = deg.reshape(NPAD, 1)

    wmlp1, bmlp1 = _eff_w(params["out_mlp"][0])
    wmlp2, bmlp2 = _eff_w(params["out_mlp"][1])
    rst = _stage_e(mx, sm, deg, s, wn.T, bn[None, :], wn2.T, bn2[None, :],
                   wmlp1.T, bmlp1[None, :], wmlp2.T, bmlp2[None, :])
    return rst
